# Initial kernel scaffold; baseline (speedup 1.0000x reference)
import functools

import jax
import jax.numpy as jnp
from jax import lax
from jax.experimental import pallas as pl
from jax.experimental.pallas import tpu as pltpu
from jax.experimental.pallas import tpu_sc as plsc

N = 10000
E = 320000
H = 128
DE = 16

NC = 2
NS = 16
NW = NC * NS
EPW = E // NW
C = 80
NCHUNK = EPW // C
RPS = N // NS

_F32 = jnp.float32


def _sc_mesh():
    return plsc.VectorSubcoreMesh(core_axis_name="c", subcore_axis_name="s")


def _seg_sum(table, src, dst, z128, z16, ones_c, with_deg):
    out_type = [jax.ShapeDtypeStruct((2 * N, H), _F32)]
    if with_deg:
        out_type.append(jax.ShapeDtypeStruct((2 * N, DE), _F32))

    scratch = [
        pltpu.VMEM((C,), jnp.int32),
        pltpu.VMEM((C,), jnp.int32),
        pltpu.VMEM((C, H), _F32),
        pltpu.VMEM_SHARED((N, H), _F32),
        pltpu.SemaphoreType.DMA,
    ]
    if with_deg:
        scratch += [
            pltpu.VMEM((C, DE), _F32),
            pltpu.VMEM_SHARED((N, DE), _F32),
        ]

    @functools.partial(
        pl.kernel,
        out_type=out_type,
        mesh=_sc_mesh(),
        scratch_types=scratch,
    )
    def k(*refs):
        if with_deg:
            (x_hbm, src_hbm, dst_hbm, z128_hbm, z16_hbm, ones_hbm,
             agg_hbm, deg_hbm,
             src_v, dst_v, rows_v, acc_sh, sem, ones_v, dacc_sh) = refs
        else:
            (x_hbm, src_hbm, dst_hbm, z128_hbm,
             agg_hbm,
             src_v, dst_v, rows_v, acc_sh, sem) = refs
        cid = lax.axis_index("c")
        sid = lax.axis_index("s")
        wid = sid * NC + cid
        r0 = sid * RPS
        pltpu.sync_copy(z128_hbm.at[pl.ds(r0, RPS)], acc_sh.at[pl.ds(r0, RPS)])
        if with_deg:
            pltpu.sync_copy(z16_hbm.at[pl.ds(r0, RPS)], dacc_sh.at[pl.ds(r0, RPS)])
            pltpu.sync_copy(ones_hbm, ones_v)
        plsc.subcore_barrier()

        base = wid * EPW

        def body(i, carry):
            off = base + i * C
            pltpu.sync_copy(src_hbm.at[pl.ds(off, C)], src_v)
            pltpu.sync_copy(dst_hbm.at[pl.ds(off, C)], dst_v)
            pltpu.async_copy(x_hbm.at[src_v], rows_v, sem).wait()
            pltpu.sync_copy(rows_v, acc_sh.at[dst_v], add=True)
            if with_deg:
                pltpu.sync_copy(ones_v, dacc_sh.at[dst_v], add=True)
            return carry

        lax.fori_loop(0, NCHUNK, body, 0)
        plsc.subcore_barrier()
        out0 = cid * N + r0
        pltpu.sync_copy(acc_sh.at[pl.ds(r0, RPS)], agg_hbm.at[pl.ds(out0, RPS)])
        if with_deg:
            pltpu.sync_copy(dacc_sh.at[pl.ds(r0, RPS)], deg_hbm.at[pl.ds(out0, RPS)])

    if with_deg:
        return k(table, src, dst, z128, z16, ones_c)
    return k(table, src, dst, z128)


def _edge_combine(a_src, a_dst, src, dst):
    @functools.partial(
        pl.kernel,
        out_type=jax.ShapeDtypeStruct((E, H), _F32),
        mesh=_sc_mesh(),
        scratch_types=[
            pltpu.VMEM((C,), jnp.int32),
            pltpu.VMEM((C,), jnp.int32),
            pltpu.VMEM((C, H), _F32),
            pltpu.VMEM((C, H), _F32),
            pltpu.SemaphoreType.DMA,
            pltpu.SemaphoreType.DMA,
        ],
    )
    def k(asrc_hbm, adst_hbm, src_hbm, dst_hbm, out_hbm,
          src_v, dst_v, r1_v, r2_v, sem1, sem2):
        cid = lax.axis_index("c")
        sid = lax.axis_index("s")
        wid = sid * NC + cid
        base = wid * EPW

        def body(i, carry):
            off = base + i * C
            pltpu.sync_copy(src_hbm.at[pl.ds(off, C)], src_v)
            pltpu.sync_copy(dst_hbm.at[pl.ds(off, C)], dst_v)
            cp1 = pltpu.async_copy(asrc_hbm.at[src_v], r1_v, sem1)
            cp2 = pltpu.async_copy(adst_hbm.at[dst_v], r2_v, sem2)
            cp1.wait()
            cp2.wait()

            def row(j, c2):
                for kk in range(H // 16):
                    sl = pl.ds(kk * 16, 16)
                    r1_v[j, sl] = r1_v[j, sl] + r2_v[j, sl]
                return c2

            lax.fori_loop(0, C, row, 0)
            pltpu.sync_copy(r1_v, out_hbm.at[pl.ds(off, C)])
            return carry

        lax.fori_loop(0, NCHUNK, body, 0)

    return k(a_src, a_dst, src, dst)


_RB = 1000
_NBLK = N // _RB


def _dotT(a, b):
    return lax.dot_general(a, b, (((1,), (1,)), ((), ())),
                           preferred_element_type=_F32)


def _sage_block(a0, a1, d0, d1, xin, wl, bl, wr):
    agg = a0[...] + a1[...]
    deg = d0[...][:, 0:1] + d1[...][:, 0:1]
    mean = agg / jnp.maximum(deg, 1.0)
    h = _dotT(mean, wl[...]) + bl[...][None, :] + _dotT(xin[...], wr[...])
    return jnp.maximum(h, 0.0)


def _t1_body(a0, a1, d0, d1, x_ref, wl, bl, wr, out_ref):
    out_ref[...] = _sage_block(a0, a1, d0, d1, x_ref, wl, bl, wr)


def _t2_body(a0, a1, d0, d1, h1_ref, wl, bl, wr, ws, wd, asrc_ref, adst_ref):
    h2 = _sage_block(a0, a1, d0, d1, h1_ref, wl, bl, wr)
    asrc_ref[...] = _dotT(h2, ws[...])
    adst_ref[...] = _dotT(h2, wd[...])


def _node_specs():
    return [
        pl.BlockSpec((_RB, H), lambda i: (i, 0)),
        pl.BlockSpec((_RB, H), lambda i: (i + _NBLK, 0)),
        pl.BlockSpec((_RB, DE), lambda i: (i, 0)),
        pl.BlockSpec((_RB, DE), lambda i: (i + _NBLK, 0)),
        pl.BlockSpec((_RB, H), lambda i: (i, 0)),
        pl.BlockSpec((H, H), lambda i: (0, 0)),
        pl.BlockSpec((H,), lambda i: (0,)),
        pl.BlockSpec((H, H), lambda i: (0, 0)),
    ]


def _t1(aggp, degp, x, wl, bl, wr):
    return pl.pallas_call(
        _t1_body,
        grid=(_NBLK,),
        in_specs=_node_specs(),
        out_specs=pl.BlockSpec((_RB, H), lambda i: (i, 0)),
        out_shape=jax.ShapeDtypeStruct((N, H), _F32),
    )(aggp, aggp, degp, degp, x, wl, bl, wr)


def _t2(aggp, degp, h1, wl, bl, wr, ws, wd):
    return pl.pallas_call(
        _t2_body,
        grid=(_NBLK,),
        in_specs=_node_specs() + [
            pl.BlockSpec((H, H), lambda i: (0, 0)),
            pl.BlockSpec((H, H), lambda i: (0, 0)),
        ],
        out_specs=[
            pl.BlockSpec((_RB, H), lambda i: (i, 0)),
            pl.BlockSpec((_RB, H), lambda i: (i, 0)),
        ],
        out_shape=[
            jax.ShapeDtypeStruct((N, H), _F32),
            jax.ShapeDtypeStruct((N, H), _F32),
        ],
    )(aggp, aggp, degp, degp, h1, wl, bl, wr, ws, wd)


_EB = 6400
_EBLK = E // _EB


def _t3_body(comb_ref, attr_ref, wattr_ref, bm1_ref, wm2_ref, bm2_ref, out_ref):
    ap = _dotT(attr_ref[...], wattr_ref[...])
    pre = comb_ref[...] + ap + bm1_ref[...][None, :]
    r = jnp.maximum(pre, 0.0)
    res = _dotT(wm2_ref[...], r)
    res = res + bm2_ref[...][:, None]
    out_ref[...] = jnp.reshape(res, (res.shape[1],))


def _t3(comb, edge_attr, wattr, bm1, wm2, bm2):
    return pl.pallas_call(
        _t3_body,
        grid=(_EBLK,),
        in_specs=[
            pl.BlockSpec((_EB, H), lambda i: (i, 0)),
            pl.BlockSpec((_EB, DE), lambda i: (i, 0)),
            pl.BlockSpec((H, DE), lambda i: (0, 0)),
            pl.BlockSpec((H,), lambda i: (0,)),
            pl.BlockSpec((1, H), lambda i: (0, 0)),
            pl.BlockSpec((1,), lambda i: (0,)),
        ],
        out_specs=pl.BlockSpec((_EB,), lambda i: (i,)),
        out_shape=jax.ShapeDtypeStruct((E,), _F32),
    )(comb, edge_attr, wattr, bm1, wm2, bm2)


def kernel(x, edge_index, edge_attr, W1l, b1l, W1r, W2l, b2l, W2r,
           Wm1, bm1, Wm2, bm2):
    src = edge_index[0].astype(jnp.int32)
    dst = edge_index[1].astype(jnp.int32)
    z128 = jnp.zeros((N, H), _F32)
    z16 = jnp.zeros((N, DE), _F32)
    ones_c = jnp.ones((C, DE), _F32)

    aggp, degp = _seg_sum(x, src, dst, z128, z16, ones_c, with_deg=True)
    h1 = _t1(aggp, degp, x, W1l, b1l, W1r)
    (agg2p,) = _seg_sum(h1, src, dst, z128, None, None, with_deg=False)
    a_src, a_dst = _t2(agg2p, degp, h1, W2l, b2l, W2r,
                       Wm1[:, :H], Wm1[:, H:2 * H])
    comb = _edge_combine(a_src, a_dst, src, dst)
    return _t3(comb, edge_attr, Wm1[:, 2 * H:], bm1, Wm2, bm2)


# speedup vs baseline: 3.7184x; 3.7184x over previous
import functools

import jax
import jax.numpy as jnp
from jax import lax
from jax.experimental import pallas as pl
from jax.experimental.pallas import tpu as pltpu
from jax.experimental.pallas import tpu_sc as plsc

N = 10000
E = 320000
H = 128
DE = 16

NC = 2
NS = 16
NW = NC * NS
EPW = E // NW
C = 80
NCHUNK = EPW // C
RPS_STEP = 624
RPS_SZ = 640

_F32 = jnp.float32


def _sc_mesh():
    return plsc.VectorSubcoreMesh(core_axis_name="c", subcore_axis_name="s")


_NZC = RPS_SZ // C


def _fill_rows(ref, rows, groups, val):
    def body(j, carry):
        for g in range(groups):
            ref[j, pl.ds(g * 16, 16)] = jnp.full((16,), val, _F32)
        return carry

    lax.fori_loop(0, rows, body, 0)


DROWS = 80


def _seg_sum(table, src, dst, with_deg):
    out_type = [jax.ShapeDtypeStruct((2 * N, H), _F32)]
    if with_deg:
        out_type.append(jax.ShapeDtypeStruct((NW, DROWS * H), _F32))

    scratch = [
        pltpu.VMEM((C,), jnp.int32),
        pltpu.VMEM((C,), jnp.int32),
        pltpu.VMEM((C, H), _F32),
        pltpu.VMEM_SHARED((N, H), _F32),
        pltpu.SemaphoreType.DMA,
    ]
    if with_deg:
        scratch.append(pltpu.VMEM((DROWS * H + 16,), _F32))

    @functools.partial(
        pl.kernel,
        out_type=out_type,
        mesh=_sc_mesh(),
        scratch_types=scratch,
    )
    def k(*refs):
        if with_deg:
            (x_hbm, src_hbm, dst_hbm,
             agg_hbm, deg_hbm,
             src_v, dst_v, rows_v, acc_sh, sem, dloc_v) = refs
        else:
            (x_hbm, src_hbm, dst_hbm,
             agg_hbm,
             src_v, dst_v, rows_v, acc_sh, sem) = refs
        cid = lax.axis_index("c")
        sid = lax.axis_index("s")
        wid = sid * NC + cid
        r0 = sid * RPS_STEP
        _fill_rows(rows_v, C, H // 16, 0.0)
        for j in range(_NZC):
            pltpu.sync_copy(rows_v, acc_sh.at[pl.ds(r0 + j * C, C)])
        if with_deg:
            def zbody(j, carry):
                dloc_v[pl.ds(j * 16, 16)] = jnp.zeros((16,), _F32)
                return carry
            lax.fori_loop(0, DROWS * H // 16 + 1, zbody, 0)
        plsc.subcore_barrier()

        base = wid * EPW
        onehot0 = jnp.where(lax.iota(jnp.int32, 16) == 0, 1.0, 0.0)

        def body(i, carry):
            off = base + i * C
            pltpu.sync_copy(src_hbm.at[pl.ds(off, C)], src_v)
            pltpu.sync_copy(dst_hbm.at[pl.ds(off, C)], dst_v)
            pltpu.async_copy(x_hbm.at[src_v], rows_v, sem).wait()
            pltpu.sync_copy(rows_v, acc_sh.at[dst_v], add=True)
            if with_deg:
                for g in range(C // 16):
                    dvec = dst_v[pl.ds(g * 16, 16)]
                    for j in range(16):
                        dd = dvec[j]
                        cur = dloc_v[pl.ds(dd, 16)]
                        dloc_v[pl.ds(dd, 16)] = cur + onehot0
            return carry

        lax.fori_loop(0, NCHUNK, body, 0)
        plsc.subcore_barrier()
        out0 = cid * N + r0
        for j in range(_NZC):
            pltpu.sync_copy(acc_sh.at[pl.ds(r0 + j * C, C)], rows_v)
            pltpu.sync_copy(rows_v, agg_hbm.at[pl.ds(out0 + j * C, C)])
        if with_deg:
            pltpu.sync_copy(dloc_v.at[pl.ds(0, DROWS * H)], deg_hbm.at[wid])

    if with_deg:
        return k(table, src, dst)
    return k(table, src, dst)


def _edge_combine(a_src, a_dst, src, dst):
    @functools.partial(
        pl.kernel,
        out_type=jax.ShapeDtypeStruct((E, H), _F32),
        mesh=_sc_mesh(),
        scratch_types=[
            pltpu.VMEM((C,), jnp.int32),
            pltpu.VMEM((C,), jnp.int32),
            pltpu.VMEM((C, H), _F32),
            pltpu.VMEM((C, H), _F32),
            pltpu.SemaphoreType.DMA,
            pltpu.SemaphoreType.DMA,
        ],
    )
    def k(asrc_hbm, adst_hbm, src_hbm, dst_hbm, out_hbm,
          src_v, dst_v, r1_v, r2_v, sem1, sem2):
        cid = lax.axis_index("c")
        sid = lax.axis_index("s")
        wid = sid * NC + cid
        base = wid * EPW

        def body(i, carry):
            off = base + i * C
            pltpu.sync_copy(src_hbm.at[pl.ds(off, C)], src_v)
            pltpu.sync_copy(dst_hbm.at[pl.ds(off, C)], dst_v)
            cp1 = pltpu.async_copy(asrc_hbm.at[src_v], r1_v, sem1)
            cp2 = pltpu.async_copy(adst_hbm.at[dst_v], r2_v, sem2)
            cp1.wait()
            cp2.wait()

            def row(j, c2):
                for kk in range(H // 16):
                    sl = pl.ds(kk * 16, 16)
                    r1_v[j, sl] = r1_v[j, sl] + r2_v[j, sl]
                return c2

            lax.fori_loop(0, C, row, 0)
            pltpu.sync_copy(r1_v, out_hbm.at[pl.ds(off, C)])
            return carry

        lax.fori_loop(0, NCHUNK, body, 0)

    return k(a_src, a_dst, src, dst)


_RB = 1024
_NBLK = (N + _RB - 1) // _RB
_DR = _RB // H


def _dotT(a, b):
    return lax.dot_general(a, b, (((1,), (1,)), ((), ())),
                           preferred_element_type=_F32)


def _dot(a, b):
    return jnp.dot(a, b, preferred_element_type=_F32)


def _sage_block(a0, a1, dw, xin, wl, bl, wr, lm, mm, ones_h):
    deg8 = jnp.sum(dw[...], axis=0)
    recip8 = 1.0 / jnp.maximum(deg8, 1.0)
    rbcast = _dot(_dot(lm[...], recip8) * mm[...], ones_h[...])
    mean = (a0[...] + a1[...]) * rbcast
    h = _dotT(mean, wl[...]) + bl[...][None, :] + _dotT(xin[...], wr[...])
    return jnp.maximum(h, 0.0)


def _t1_body(a0, a1, dw, x_ref, wl, bl, wr, lm, mm, ones_h, out_ref):
    out_ref[...] = _sage_block(a0, a1, dw, x_ref, wl, bl, wr, lm, mm, ones_h)


def _t2_body(a0, a1, dw, h1_ref, wl, bl, wr, lm, mm, ones_h,
             ws, wd, asrc_ref, adst_ref):
    h2 = _sage_block(a0, a1, dw, h1_ref, wl, bl, wr, lm, mm, ones_h)
    asrc_ref[...] = _dotT(h2, ws[...])
    adst_ref[...] = _dotT(h2, wd[...])


def _node_specs():
    return [
        pl.BlockSpec((_RB, H), lambda i: (i, 0)),
        pl.BlockSpec((_RB, H), lambda i: (i, 0)),
        pl.BlockSpec((NW, _DR, H), lambda i: (0, i, 0)),
        pl.BlockSpec((_RB, H), lambda i: (i, 0)),
        pl.BlockSpec((H, H), lambda i: (0, 0)),
        pl.BlockSpec((H,), lambda i: (0,)),
        pl.BlockSpec((H, H), lambda i: (0, 0)),
        pl.BlockSpec((_RB, _DR), lambda i: (0, 0)),
        pl.BlockSpec((_RB, H), lambda i: (0, 0)),
        pl.BlockSpec((H, H), lambda i: (0, 0)),
    ]


def _unpack_consts():
    j = jnp.arange(_RB)
    lm = (j[:, None] // H == jnp.arange(_DR)[None, :]).astype(_F32)
    mm = (j[:, None] % H == jnp.arange(H)[None, :]).astype(_F32)
    ones_h = jnp.ones((H, H), _F32)
    return lm, mm, ones_h


def _t1(agg0, agg1, degw, x, wl, bl, wr):
    lm, mm, ones_h = _unpack_consts()
    return pl.pallas_call(
        _t1_body,
        grid=(_NBLK,),
        in_specs=_node_specs(),
        out_specs=pl.BlockSpec((_RB, H), lambda i: (i, 0)),
        out_shape=jax.ShapeDtypeStruct((N, H), _F32),
    )(agg0, agg1, degw, x, wl, bl, wr, lm, mm, ones_h)


def _t2(agg0, agg1, degw, h1, wl, bl, wr, ws, wd):
    lm, mm, ones_h = _unpack_consts()
    return pl.pallas_call(
        _t2_body,
        grid=(_NBLK,),
        in_specs=_node_specs() + [
            pl.BlockSpec((H, H), lambda i: (0, 0)),
            pl.BlockSpec((H, H), lambda i: (0, 0)),
        ],
        out_specs=[
            pl.BlockSpec((_RB, H), lambda i: (i, 0)),
            pl.BlockSpec((_RB, H), lambda i: (i, 0)),
        ],
        out_shape=[
            jax.ShapeDtypeStruct((N, H), _F32),
            jax.ShapeDtypeStruct((N, H), _F32),
        ],
    )(agg0, agg1, degw, h1, wl, bl, wr, lm, mm, ones_h, ws, wd)


_EB = 4096
_EBLK = (E + _EB - 1) // _EB


def _t3_body(comb_ref, attr_ref, wattr_ref, bm1_ref, wm2_ref, bm2_ref, out_ref):
    ap = _dotT(attr_ref[...], wattr_ref[...])
    pre = comb_ref[...] + ap + bm1_ref[...][None, :]
    r = jnp.maximum(pre, 0.0)
    res = _dotT(wm2_ref[...], r)
    res = res + bm2_ref[...][:, None]
    out_ref[...] = jnp.reshape(res, (res.shape[1],))


def _t3(comb, edge_attr, wattr, bm1, wm2, bm2):
    return pl.pallas_call(
        _t3_body,
        grid=(_EBLK,),
        in_specs=[
            pl.BlockSpec((_EB, H), lambda i: (i, 0)),
            pl.BlockSpec((_EB, DE), lambda i: (i, 0)),
            pl.BlockSpec((H, DE), lambda i: (0, 0)),
            pl.BlockSpec((H,), lambda i: (0,)),
            pl.BlockSpec((1, H), lambda i: (0, 0)),
            pl.BlockSpec((1,), lambda i: (0,)),
        ],
        out_specs=pl.BlockSpec((_EB,), lambda i: (i,)),
        out_shape=jax.ShapeDtypeStruct((E,), _F32),
    )(comb, edge_attr, wattr, bm1, wm2, bm2)


def kernel(x, edge_index, edge_attr, W1l, b1l, W1r, W2l, b2l, W2r,
           Wm1, bm1, Wm2, bm2):
    src = edge_index[0].astype(jnp.int32)
    dst = edge_index[1].astype(jnp.int32)

    aggp, degp = _seg_sum(x, src, dst, with_deg=True)
    degw = degp.reshape(NW, DROWS, H)
    h1 = _t1(aggp[:N], aggp[N:], degw, x, W1l, b1l, W1r)
    (agg2p,) = _seg_sum(h1, src, dst, with_deg=False)
    a_src, a_dst = _t2(agg2p[:N], agg2p[N:], degw, h1, W2l, b2l, W2r,
                       Wm1[:, :H], Wm1[:, H:2 * H])
    comb = _edge_combine(a_src, a_dst, src, dst)
    return _t3(comb, edge_attr, Wm1[:, 2 * H:], bm1, Wm2, bm2)


# speedup vs baseline: 6.0671x; 1.6317x over previous
import functools

import jax
import jax.numpy as jnp
from jax import lax
from jax.experimental import pallas as pl
from jax.experimental.pallas import tpu as pltpu
from jax.experimental.pallas import tpu_sc as plsc

N = 10000
E = 320000
H = 128
DE = 16

NC = 2
NS = 16
NW = NC * NS
EPW = E // NW
C = 80
NCHUNK = EPW // C
RPS_STEP = 624
RPS_SZ = 640

_F32 = jnp.float32


def _sc_mesh():
    return plsc.VectorSubcoreMesh(core_axis_name="c", subcore_axis_name="s")


_NZC = RPS_SZ // C


def _fill_rows(ref, rows, groups, val):
    def body(j, carry):
        for g in range(groups):
            ref[j, pl.ds(g * 16, 16)] = jnp.full((16,), val, _F32)
        return carry

    lax.fori_loop(0, rows, body, 0)


DROWS = 80
NBUF = 5

CS = 40
NCH_S = EPW // CS
NOUT_S = NCH_S // NBUF
CE = 80
NCH_E = EPW // CE
NOUT_E = NCH_E // NBUF

_OFFS_S = (0, 16, 24)
_OFFS_E = (0, 16, 32, 48, 64)


def _regcopy_chunk(dst2d, j, src1d, off, offs):
    for g in offs:
        dst2d[j, pl.ds(g, 16)] = src1d[pl.ds(off + g, 16)]


def _seg_sum(table, src, dst, with_deg):
    out_type = [jax.ShapeDtypeStruct((2 * N, H), _F32)]
    if with_deg:
        out_type.append(jax.ShapeDtypeStruct((NW, DROWS * H), _F32))

    scratch = [
        pltpu.VMEM((NBUF * CS,), jnp.int32),
        pltpu.VMEM((NBUF * CS,), jnp.int32),
        pltpu.VMEM((NBUF, CS), jnp.int32),
        pltpu.VMEM((NBUF, CS), jnp.int32),
        pltpu.VMEM((NBUF, CS, H), _F32),
        pltpu.VMEM_SHARED((N, H), _F32),
    ] + [pltpu.SemaphoreType.DMA] * (2 * NBUF)
    if with_deg:
        scratch.append(pltpu.VMEM((DROWS * H + 16,), _F32))

    @functools.partial(
        pl.kernel,
        out_type=out_type,
        mesh=_sc_mesh(),
        scratch_types=scratch,
    )
    def k(*refs):
        if with_deg:
            (x_hbm, src_hbm, dst_hbm, agg_hbm, deg_hbm,
             sblk_v, dblk_v, srcc_v, dstc_v, rows_v, acc_sh,
             *sems, dloc_v) = refs
        else:
            (x_hbm, src_hbm, dst_hbm, agg_hbm,
             sblk_v, dblk_v, srcc_v, dstc_v, rows_v, acc_sh,
             *sems) = refs
        gsem = sems[:NBUF]
        ssem = sems[NBUF:2 * NBUF]
        cid = lax.axis_index("c")
        sid = lax.axis_index("s")
        wid = sid * NC + cid
        r0 = sid * RPS_STEP
        _fill_rows(rows_v.at[0], CS, H // 16, 0.0)
        for j in range(RPS_SZ // CS):
            pltpu.sync_copy(rows_v.at[0], acc_sh.at[pl.ds(r0 + j * CS, CS)])
        if with_deg:
            def zbody(j, carry):
                dloc_v[pl.ds(j * 16, 16)] = jnp.zeros((16,), _F32)
                return carry
            lax.fori_loop(0, DROWS * H // 16 + 1, zbody, 0)
        plsc.subcore_barrier()

        base = wid * EPW
        onehot0 = jnp.where(lax.iota(jnp.int32, 16) == 0, 1.0, 0.0)

        pltpu.sync_copy(src_hbm.at[pl.ds(base, NBUF * CS)], sblk_v)
        pltpu.sync_copy(dst_hbm.at[pl.ds(base, NBUF * CS)], dblk_v)
        for j in range(NBUF):
            _regcopy_chunk(srcc_v, j, sblk_v, j * CS, _OFFS_S)
            _regcopy_chunk(dstc_v, j, dblk_v, j * CS, _OFFS_S)
            pltpu.async_copy(x_hbm.at[srcc_v.at[j]], rows_v.at[j], gsem[j])

        def outer(o, carry):
            for j in range(NBUF):
                pltpu.make_async_copy(
                    x_hbm.at[srcc_v.at[j]], rows_v.at[j], gsem[j]).wait()
                pltpu.async_copy(rows_v.at[j], acc_sh.at[dstc_v.at[j]],
                                 ssem[j], add=True)
            if with_deg:
                for jc in range(NBUF):
                    for g2, lanes in ((0, range(16)), (16, range(16)),
                                      (24, range(8, 16))):
                        dvec = dstc_v[jc, pl.ds(g2, 16)]
                        for l in lanes:
                            dd = dvec[l]
                            cur = dloc_v[pl.ds(dd, 16)]
                            dloc_v[pl.ds(dd, 16)] = cur + onehot0
            @pl.when(o < NOUT_S - 1)
            def _next():
                nb = base + (o + 1) * NBUF * CS
                pltpu.sync_copy(src_hbm.at[pl.ds(nb, NBUF * CS)], sblk_v)
                pltpu.sync_copy(dst_hbm.at[pl.ds(nb, NBUF * CS)], dblk_v)
                for j in range(NBUF):
                    pltpu.make_async_copy(
                        rows_v.at[j], acc_sh.at[dstc_v.at[j]], ssem[j]).wait()
                    _regcopy_chunk(srcc_v, j, sblk_v, j * CS, _OFFS_S)
                    _regcopy_chunk(dstc_v, j, dblk_v, j * CS, _OFFS_S)
                    pltpu.async_copy(x_hbm.at[srcc_v.at[j]], rows_v.at[j],
                                     gsem[j])
            return carry

        lax.fori_loop(0, NOUT_S, outer, 0)
        for j in range(NBUF):
            pltpu.make_async_copy(
                rows_v.at[j], acc_sh.at[dstc_v.at[j]], ssem[j]).wait()
        plsc.subcore_barrier()
        out0 = cid * N + r0
        for j in range(RPS_SZ // CS):
            pltpu.sync_copy(acc_sh.at[pl.ds(r0 + j * CS, CS)], rows_v.at[0])
            pltpu.sync_copy(rows_v.at[0], agg_hbm.at[pl.ds(out0 + j * CS, CS)])
        if with_deg:
            pltpu.sync_copy(dloc_v.at[pl.ds(0, DROWS * H)], deg_hbm.at[wid])

    if with_deg:
        return k(table, src, dst)
    return k(table, src, dst)


def _edge_combine(a_src, a_dst, src, dst):
    @functools.partial(
        pl.kernel,
        out_type=jax.ShapeDtypeStruct((E, H), _F32),
        mesh=_sc_mesh(),
        scratch_types=[
            pltpu.VMEM((EPW,), jnp.int32),
            pltpu.VMEM((EPW,), jnp.int32),
            pltpu.VMEM((NBUF, CE), jnp.int32),
            pltpu.VMEM((NBUF, CE), jnp.int32),
            pltpu.VMEM((NBUF, CE, H), _F32),
            pltpu.VMEM((NBUF, CE, H), _F32),
        ] + [pltpu.SemaphoreType.DMA] * (3 * NBUF),
    )
    def k(asrc_hbm, adst_hbm, src_hbm, dst_hbm, out_hbm,
          sidx_v, didx_v, srcc_v, dstc_v, r1_v, r2_v, *sems):
        g1sem = sems[:NBUF]
        g2sem = sems[NBUF:2 * NBUF]
        wsem = sems[2 * NBUF:3 * NBUF]
        cid = lax.axis_index("c")
        sid = lax.axis_index("s")
        wid = sid * NC + cid
        base = wid * EPW

        pltpu.sync_copy(src_hbm.at[pl.ds(base, EPW)], sidx_v)
        pltpu.sync_copy(dst_hbm.at[pl.ds(base, EPW)], didx_v)
        for j in range(NBUF):
            _regcopy_chunk(srcc_v, j, sidx_v, j * CE, _OFFS_E)
            _regcopy_chunk(dstc_v, j, didx_v, j * CE, _OFFS_E)
            pltpu.async_copy(asrc_hbm.at[srcc_v.at[j]], r1_v.at[j], g1sem[j])
            pltpu.async_copy(adst_hbm.at[dstc_v.at[j]], r2_v.at[j], g2sem[j])

        def outer(o, carry):
            for j in range(NBUF):
                off = base + (o * NBUF + j) * CE
                pltpu.make_async_copy(
                    asrc_hbm.at[srcc_v.at[j]], r1_v.at[j], g1sem[j]).wait()
                pltpu.make_async_copy(
                    adst_hbm.at[dstc_v.at[j]], r2_v.at[j], g2sem[j]).wait()

                def row(jr, c2, j=j):
                    for kk in range(H // 16):
                        sl = pl.ds(kk * 16, 16)
                        r1_v[j, jr, sl] = r1_v[j, jr, sl] + r2_v[j, jr, sl]
                    return c2

                lax.fori_loop(0, CE, row, 0)
                pltpu.async_copy(r1_v.at[j], out_hbm.at[pl.ds(off, CE)],
                                 wsem[j])

                @pl.when(o < NOUT_E - 1)
                def _launch2(j=j):
                    nxt = ((o + 1) * NBUF + j) * CE
                    _regcopy_chunk(dstc_v, j, didx_v, nxt, _OFFS_E)
                    pltpu.async_copy(adst_hbm.at[dstc_v.at[j]], r2_v.at[j],
                                     g2sem[j])
            for j in range(NBUF):
                @pl.when(o < NOUT_E - 1)
                def _launch1(j=j):
                    off = base + (o * NBUF + j) * CE
                    pltpu.make_async_copy(
                        r1_v.at[j], out_hbm.at[pl.ds(off, CE)], wsem[j]).wait()
                    nxt = ((o + 1) * NBUF + j) * CE
                    _regcopy_chunk(srcc_v, j, sidx_v, nxt, _OFFS_E)
                    pltpu.async_copy(asrc_hbm.at[srcc_v.at[j]], r1_v.at[j],
                                     g1sem[j])
            return carry

        lax.fori_loop(0, NOUT_E, outer, 0)
        for j in range(NBUF):
            off = base + ((NOUT_E - 1) * NBUF + j) * CE
            pltpu.make_async_copy(
                r1_v.at[j], out_hbm.at[pl.ds(off, CE)], wsem[j]).wait()

    return k(a_src, a_dst, src, dst)


_RB = 1024
_NBLK = (N + _RB - 1) // _RB
_DR = _RB // H


def _dotT(a, b):
    return lax.dot_general(a, b, (((1,), (1,)), ((), ())),
                           preferred_element_type=_F32)


def _dot(a, b):
    return jnp.dot(a, b, preferred_element_type=_F32)


def _sage_block(a0, a1, dw, xin, wl, bl, wr, lm, mm, ones_h):
    deg8 = jnp.sum(dw[...], axis=0)
    recip8 = 1.0 / jnp.maximum(deg8, 1.0)
    rbcast = _dot(_dot(lm[...], recip8) * mm[...], ones_h[...])
    mean = (a0[...] + a1[...]) * rbcast
    h = _dotT(mean, wl[...]) + bl[...][None, :] + _dotT(xin[...], wr[...])
    return jnp.maximum(h, 0.0)


def _t1_body(a0, a1, dw, x_ref, wl, bl, wr, lm, mm, ones_h, out_ref):
    out_ref[...] = _sage_block(a0, a1, dw, x_ref, wl, bl, wr, lm, mm, ones_h)


def _t2_body(a0, a1, dw, h1_ref, wl, bl, wr, lm, mm, ones_h,
             ws, wd, asrc_ref, adst_ref):
    h2 = _sage_block(a0, a1, dw, h1_ref, wl, bl, wr, lm, mm, ones_h)
    asrc_ref[...] = _dotT(h2, ws[...])
    adst_ref[...] = _dotT(h2, wd[...])


def _node_specs():
    return [
        pl.BlockSpec((_RB, H), lambda i: (i, 0)),
        pl.BlockSpec((_RB, H), lambda i: (i, 0)),
        pl.BlockSpec((NW, _DR, H), lambda i: (0, i, 0)),
        pl.BlockSpec((_RB, H), lambda i: (i, 0)),
        pl.BlockSpec((H, H), lambda i: (0, 0)),
        pl.BlockSpec((H,), lambda i: (0,)),
        pl.BlockSpec((H, H), lambda i: (0, 0)),
        pl.BlockSpec((_RB, _DR), lambda i: (0, 0)),
        pl.BlockSpec((_RB, H), lambda i: (0, 0)),
        pl.BlockSpec((H, H), lambda i: (0, 0)),
    ]


def _unpack_consts():
    j = jnp.arange(_RB)
    lm = (j[:, None] // H == jnp.arange(_DR)[None, :]).astype(_F32)
    mm = (j[:, None] % H == jnp.arange(H)[None, :]).astype(_F32)
    ones_h = jnp.ones((H, H), _F32)
    return lm, mm, ones_h


def _t1(agg0, agg1, degw, x, wl, bl, wr):
    lm, mm, ones_h = _unpack_consts()
    return pl.pallas_call(
        _t1_body,
        grid=(_NBLK,),
        in_specs=_node_specs(),
        out_specs=pl.BlockSpec((_RB, H), lambda i: (i, 0)),
        out_shape=jax.ShapeDtypeStruct((N, H), _F32),
    )(agg0, agg1, degw, x, wl, bl, wr, lm, mm, ones_h)


def _t2(agg0, agg1, degw, h1, wl, bl, wr, ws, wd):
    lm, mm, ones_h = _unpack_consts()
    return pl.pallas_call(
        _t2_body,
        grid=(_NBLK,),
        in_specs=_node_specs() + [
            pl.BlockSpec((H, H), lambda i: (0, 0)),
            pl.BlockSpec((H, H), lambda i: (0, 0)),
        ],
        out_specs=[
            pl.BlockSpec((_RB, H), lambda i: (i, 0)),
            pl.BlockSpec((_RB, H), lambda i: (i, 0)),
        ],
        out_shape=[
            jax.ShapeDtypeStruct((N, H), _F32),
            jax.ShapeDtypeStruct((N, H), _F32),
        ],
    )(agg0, agg1, degw, h1, wl, bl, wr, lm, mm, ones_h, ws, wd)


_EB = 4096
_EBLK = (E + _EB - 1) // _EB


def _t3_body(comb_ref, attr_ref, wattr_ref, bm1_ref, wm2_ref, bm2_ref, out_ref):
    ap = _dotT(attr_ref[...], wattr_ref[...])
    pre = comb_ref[...] + ap + bm1_ref[...][None, :]
    r = jnp.maximum(pre, 0.0)
    res = _dotT(wm2_ref[...], r)
    res = res + bm2_ref[...][:, None]
    out_ref[...] = jnp.reshape(res, (res.shape[1],))


def _t3(comb, edge_attr, wattr, bm1, wm2, bm2):
    return pl.pallas_call(
        _t3_body,
        grid=(_EBLK,),
        in_specs=[
            pl.BlockSpec((_EB, H), lambda i: (i, 0)),
            pl.BlockSpec((_EB, DE), lambda i: (i, 0)),
            pl.BlockSpec((H, DE), lambda i: (0, 0)),
            pl.BlockSpec((H,), lambda i: (0,)),
            pl.BlockSpec((1, H), lambda i: (0, 0)),
            pl.BlockSpec((1,), lambda i: (0,)),
        ],
        out_specs=pl.BlockSpec((_EB,), lambda i: (i,)),
        out_shape=jax.ShapeDtypeStruct((E,), _F32),
    )(comb, edge_attr, wattr, bm1, wm2, bm2)


def kernel(x, edge_index, edge_attr, W1l, b1l, W1r, W2l, b2l, W2r,
           Wm1, bm1, Wm2, bm2):
    src = edge_index[0].astype(jnp.int32)
    dst = edge_index[1].astype(jnp.int32)

    aggp, degp = _seg_sum(x, src, dst, with_deg=True)
    degw = degp.reshape(NW, DROWS, H)
    h1 = _t1(aggp[:N], aggp[N:], degw, x, W1l, b1l, W1r)
    (agg2p,) = _seg_sum(h1, src, dst, with_deg=False)
    a_src, a_dst = _t2(agg2p[:N], agg2p[N:], degw, h1, W2l, b2l, W2r,
                       Wm1[:, :H], Wm1[:, H:2 * H])
    comb = _edge_combine(a_src, a_dst, src, dst)
    return _t3(comb, edge_attr, Wm1[:, 2 * H:], bm1, Wm2, bm2)


# speedup vs baseline: 6.1004x; 1.0055x over previous
import functools

import jax
import jax.numpy as jnp
from jax import lax
from jax.experimental import pallas as pl
from jax.experimental.pallas import tpu as pltpu
from jax.experimental.pallas import tpu_sc as plsc

N = 10000
E = 320000
H = 128
DE = 16

NC = 2
NS = 16
NW = NC * NS
EPW = E // NW
C = 80
NCHUNK = EPW // C
RPS_STEP = 624
RPS_SZ = 640

_F32 = jnp.float32


def _sc_mesh():
    return plsc.VectorSubcoreMesh(core_axis_name="c", subcore_axis_name="s")


_NZC = RPS_SZ // C


def _fill_rows(ref, rows, groups, val):
    def body(j, carry):
        for g in range(groups):
            ref[j, pl.ds(g * 16, 16)] = jnp.full((16,), val, _F32)
        return carry

    lax.fori_loop(0, rows, body, 0)


DROWS = 80
NBUF = 5

CS = 40
NCH_S = EPW // CS
NOUT_S = NCH_S // NBUF
CE = 80
NCH_E = EPW // CE
NOUT_E = NCH_E // NBUF

_OFFS_S = (0, 16, 24)
_OFFS_E = (0, 16, 32, 48, 64)


def _regcopy_chunk(dst2d, j, src1d, off, offs):
    for g in offs:
        dst2d[j, pl.ds(g, 16)] = src1d[pl.ds(off + g, 16)]


def _seg_sum(table, src, dst, with_deg):
    out_type = [jax.ShapeDtypeStruct((2 * N, H), _F32)]
    if with_deg:
        out_type.append(jax.ShapeDtypeStruct((NW, DROWS * H), _F32))

    scratch = [
        pltpu.VMEM((NBUF * CS,), jnp.int32),
        pltpu.VMEM((NBUF * CS,), jnp.int32),
        pltpu.VMEM((NBUF, CS), jnp.int32),
        pltpu.VMEM((NBUF, CS), jnp.int32),
        pltpu.VMEM((NBUF, CS, H), _F32),
        pltpu.VMEM_SHARED((N, H), _F32),
    ] + [pltpu.SemaphoreType.DMA] * (2 * NBUF)
    if with_deg:
        scratch.append(pltpu.VMEM((DROWS * H + 16,), _F32))

    @functools.partial(
        pl.kernel,
        out_type=out_type,
        mesh=_sc_mesh(),
        scratch_types=scratch,
    )
    def k(*refs):
        if with_deg:
            (x_hbm, src_hbm, dst_hbm, agg_hbm, deg_hbm,
             sblk_v, dblk_v, srcc_v, dstc_v, rows_v, acc_sh,
             *sems, dloc_v) = refs
        else:
            (x_hbm, src_hbm, dst_hbm, agg_hbm,
             sblk_v, dblk_v, srcc_v, dstc_v, rows_v, acc_sh,
             *sems) = refs
        gsem = sems[:NBUF]
        ssem = sems[NBUF:2 * NBUF]
        cid = lax.axis_index("c")
        sid = lax.axis_index("s")
        wid = sid * NC + cid
        r0 = sid * RPS_STEP
        _fill_rows(rows_v.at[0], CS, H // 16, 0.0)
        for j in range(RPS_SZ // CS):
            pltpu.sync_copy(rows_v.at[0], acc_sh.at[pl.ds(r0 + j * CS, CS)])
        if with_deg:
            def zbody(j, carry):
                dloc_v[pl.ds(j * 16, 16)] = jnp.zeros((16,), _F32)
                return carry
            lax.fori_loop(0, DROWS * H // 16 + 1, zbody, 0)
        plsc.subcore_barrier()

        base = wid * EPW
        onehot0 = jnp.where(lax.iota(jnp.int32, 16) == 0, 1.0, 0.0)

        pltpu.sync_copy(src_hbm.at[pl.ds(base, NBUF * CS)], sblk_v)
        pltpu.sync_copy(dst_hbm.at[pl.ds(base, NBUF * CS)], dblk_v)
        for j in range(NBUF):
            _regcopy_chunk(srcc_v, j, sblk_v, j * CS, _OFFS_S)
            _regcopy_chunk(dstc_v, j, dblk_v, j * CS, _OFFS_S)
            pltpu.async_copy(x_hbm.at[srcc_v.at[j]], rows_v.at[j], gsem[j])

        def outer(o, carry):
            for j in range(NBUF):
                pltpu.make_async_copy(
                    x_hbm.at[srcc_v.at[j]], rows_v.at[j], gsem[j]).wait()
                pltpu.async_copy(rows_v.at[j], acc_sh.at[dstc_v.at[j]],
                                 ssem[j], add=True)
            if with_deg:
                for jc in range(NBUF):
                    for g2, lanes in ((0, range(16)), (16, range(16)),
                                      (24, range(8, 16))):
                        dvec = dstc_v[jc, pl.ds(g2, 16)]
                        for l in lanes:
                            dd = dvec[l]
                            cur = dloc_v[pl.ds(dd, 16)]
                            dloc_v[pl.ds(dd, 16)] = cur + onehot0
            @pl.when(o < NOUT_S - 1)
            def _next():
                nb = base + (o + 1) * NBUF * CS
                pltpu.sync_copy(src_hbm.at[pl.ds(nb, NBUF * CS)], sblk_v)
                pltpu.sync_copy(dst_hbm.at[pl.ds(nb, NBUF * CS)], dblk_v)
                for j in range(NBUF):
                    pltpu.make_async_copy(
                        rows_v.at[j], acc_sh.at[dstc_v.at[j]], ssem[j]).wait()
                    _regcopy_chunk(srcc_v, j, sblk_v, j * CS, _OFFS_S)
                    _regcopy_chunk(dstc_v, j, dblk_v, j * CS, _OFFS_S)
                    pltpu.async_copy(x_hbm.at[srcc_v.at[j]], rows_v.at[j],
                                     gsem[j])
            return carry

        lax.fori_loop(0, NOUT_S, outer, 0)
        for j in range(NBUF):
            pltpu.make_async_copy(
                rows_v.at[j], acc_sh.at[dstc_v.at[j]], ssem[j]).wait()
        plsc.subcore_barrier()
        out0 = cid * N + r0
        for j in range(RPS_SZ // CS):
            pltpu.sync_copy(acc_sh.at[pl.ds(r0 + j * CS, CS)], rows_v.at[0])
            pltpu.sync_copy(rows_v.at[0], agg_hbm.at[pl.ds(out0 + j * CS, CS)])
        if with_deg:
            pltpu.sync_copy(dloc_v.at[pl.ds(0, DROWS * H)], deg_hbm.at[wid])

    if with_deg:
        return k(table, src, dst)
    return k(table, src, dst)


def _edge_combine(a_src, a_dst, src, dst):
    @functools.partial(
        pl.kernel,
        out_type=jax.ShapeDtypeStruct((E, H), _F32),
        mesh=_sc_mesh(),
        scratch_types=[
            pltpu.VMEM((EPW,), jnp.int32),
            pltpu.VMEM((EPW,), jnp.int32),
            pltpu.VMEM((NBUF, CE), jnp.int32),
            pltpu.VMEM((NBUF, CE), jnp.int32),
            pltpu.VMEM((NBUF, CE, H), _F32),
        ] + [pltpu.SemaphoreType.DMA] * (3 * NBUF),
    )
    def k(asrc_hbm, adst_hbm, src_hbm, dst_hbm, out_hbm,
          sidx_v, didx_v, srcc_v, dstc_v, r1_v, *sems):
        g1sem = sems[:NBUF]
        g2sem = sems[NBUF:2 * NBUF]
        wsem = sems[2 * NBUF:3 * NBUF]
        cid = lax.axis_index("c")
        sid = lax.axis_index("s")
        wid = sid * NC + cid
        base = wid * EPW

        pltpu.sync_copy(src_hbm.at[pl.ds(base, EPW)], sidx_v)
        pltpu.sync_copy(dst_hbm.at[pl.ds(base, EPW)], didx_v)
        for j in range(NBUF):
            _regcopy_chunk(srcc_v, j, sidx_v, j * CE, _OFFS_E)
            _regcopy_chunk(dstc_v, j, didx_v, j * CE, _OFFS_E)
            pltpu.async_copy(asrc_hbm.at[srcc_v.at[j]], r1_v.at[j], g1sem[j])

        def outer(o, carry):
            for j in range(NBUF):
                pltpu.make_async_copy(
                    asrc_hbm.at[srcc_v.at[j]], r1_v.at[j], g1sem[j]).wait()
                pltpu.async_copy(adst_hbm.at[dstc_v.at[j]], r1_v.at[j],
                                 g2sem[j], add=True)
            for j in range(NBUF):
                off = base + (o * NBUF + j) * CE
                pltpu.make_async_copy(
                    adst_hbm.at[dstc_v.at[j]], r1_v.at[j], g2sem[j]).wait()
                pltpu.async_copy(r1_v.at[j], out_hbm.at[pl.ds(off, CE)],
                                 wsem[j])
            for j in range(NBUF):
                @pl.when(o < NOUT_E - 1)
                def _launch1(j=j):
                    off = base + (o * NBUF + j) * CE
                    pltpu.make_async_copy(
                        r1_v.at[j], out_hbm.at[pl.ds(off, CE)], wsem[j]).wait()
                    nxt = ((o + 1) * NBUF + j) * CE
                    _regcopy_chunk(srcc_v, j, sidx_v, nxt, _OFFS_E)
                    _regcopy_chunk(dstc_v, j, didx_v, nxt, _OFFS_E)
                    pltpu.async_copy(asrc_hbm.at[srcc_v.at[j]], r1_v.at[j],
                                     g1sem[j])
            return carry

        lax.fori_loop(0, NOUT_E, outer, 0)
        for j in range(NBUF):
            off = base + ((NOUT_E - 1) * NBUF + j) * CE
            pltpu.make_async_copy(
                r1_v.at[j], out_hbm.at[pl.ds(off, CE)], wsem[j]).wait()

    return k(a_src, a_dst, src, dst)


_RB = 1024
_NBLK = (N + _RB - 1) // _RB
_DR = _RB // H


def _dotT(a, b):
    return lax.dot_general(a, b, (((1,), (1,)), ((), ())),
                           preferred_element_type=_F32)


def _dot(a, b):
    return jnp.dot(a, b, preferred_element_type=_F32)


def _sage_block(a0, a1, dw, xin, wl, bl, wr, lm, mm, ones_h):
    deg8 = jnp.sum(dw[...], axis=0)
    recip8 = 1.0 / jnp.maximum(deg8, 1.0)
    rbcast = _dot(_dot(lm[...], recip8) * mm[...], ones_h[...])
    mean = (a0[...] + a1[...]) * rbcast
    h = _dotT(mean, wl[...]) + bl[...][None, :] + _dotT(xin[...], wr[...])
    return jnp.maximum(h, 0.0)


def _t1_body(a0, a1, dw, x_ref, wl, bl, wr, lm, mm, ones_h, out_ref):
    out_ref[...] = _sage_block(a0, a1, dw, x_ref, wl, bl, wr, lm, mm, ones_h)


def _t2_body(a0, a1, dw, h1_ref, wl, bl, wr, lm, mm, ones_h,
             ws, wd, asrc_ref, adst_ref):
    h2 = _sage_block(a0, a1, dw, h1_ref, wl, bl, wr, lm, mm, ones_h)
    asrc_ref[...] = _dotT(h2, ws[...])
    adst_ref[...] = _dotT(h2, wd[...])


def _node_specs():
    return [
        pl.BlockSpec((_RB, H), lambda i: (i, 0)),
        pl.BlockSpec((_RB, H), lambda i: (i, 0)),
        pl.BlockSpec((NW, _DR, H), lambda i: (0, i, 0)),
        pl.BlockSpec((_RB, H), lambda i: (i, 0)),
        pl.BlockSpec((H, H), lambda i: (0, 0)),
        pl.BlockSpec((H,), lambda i: (0,)),
        pl.BlockSpec((H, H), lambda i: (0, 0)),
        pl.BlockSpec((_RB, _DR), lambda i: (0, 0)),
        pl.BlockSpec((_RB, H), lambda i: (0, 0)),
        pl.BlockSpec((H, H), lambda i: (0, 0)),
    ]


def _unpack_consts():
    j = jnp.arange(_RB)
    lm = (j[:, None] // H == jnp.arange(_DR)[None, :]).astype(_F32)
    mm = (j[:, None] % H == jnp.arange(H)[None, :]).astype(_F32)
    ones_h = jnp.ones((H, H), _F32)
    return lm, mm, ones_h


def _t1(agg0, agg1, degw, x, wl, bl, wr):
    lm, mm, ones_h = _unpack_consts()
    return pl.pallas_call(
        _t1_body,
        grid=(_NBLK,),
        in_specs=_node_specs(),
        out_specs=pl.BlockSpec((_RB, H), lambda i: (i, 0)),
        out_shape=jax.ShapeDtypeStruct((N, H), _F32),
    )(agg0, agg1, degw, x, wl, bl, wr, lm, mm, ones_h)


def _t2(agg0, agg1, degw, h1, wl, bl, wr, ws, wd):
    lm, mm, ones_h = _unpack_consts()
    return pl.pallas_call(
        _t2_body,
        grid=(_NBLK,),
        in_specs=_node_specs() + [
            pl.BlockSpec((H, H), lambda i: (0, 0)),
            pl.BlockSpec((H, H), lambda i: (0, 0)),
        ],
        out_specs=[
            pl.BlockSpec((_RB, H), lambda i: (i, 0)),
            pl.BlockSpec((_RB, H), lambda i: (i, 0)),
        ],
        out_shape=[
            jax.ShapeDtypeStruct((N, H), _F32),
            jax.ShapeDtypeStruct((N, H), _F32),
        ],
    )(agg0, agg1, degw, h1, wl, bl, wr, lm, mm, ones_h, ws, wd)


_EB = 4096
_EBLK = (E + _EB - 1) // _EB


def _t3_body(comb_ref, attr_ref, wattr_ref, bm1_ref, wm2_ref, bm2_ref, out_ref):
    ap = _dotT(attr_ref[...], wattr_ref[...])
    pre = comb_ref[...] + ap + bm1_ref[...][None, :]
    r = jnp.maximum(pre, 0.0)
    res = _dotT(wm2_ref[...], r)
    res = res + bm2_ref[...][:, None]
    out_ref[...] = jnp.reshape(res, (res.shape[1],))


def _t3(comb, edge_attr, wattr, bm1, wm2, bm2):
    return pl.pallas_call(
        _t3_body,
        grid=(_EBLK,),
        in_specs=[
            pl.BlockSpec((_EB, H), lambda i: (i, 0)),
            pl.BlockSpec((_EB, DE), lambda i: (i, 0)),
            pl.BlockSpec((H, DE), lambda i: (0, 0)),
            pl.BlockSpec((H,), lambda i: (0,)),
            pl.BlockSpec((1, H), lambda i: (0, 0)),
            pl.BlockSpec((1,), lambda i: (0,)),
        ],
        out_specs=pl.BlockSpec((_EB,), lambda i: (i,)),
        out_shape=jax.ShapeDtypeStruct((E,), _F32),
    )(comb, edge_attr, wattr, bm1, wm2, bm2)


def kernel(x, edge_index, edge_attr, W1l, b1l, W1r, W2l, b2l, W2r,
           Wm1, bm1, Wm2, bm2):
    src = edge_index[0].astype(jnp.int32)
    dst = edge_index[1].astype(jnp.int32)

    aggp, degp = _seg_sum(x, src, dst, with_deg=True)
    degw = degp.reshape(NW, DROWS, H)
    h1 = _t1(aggp[:N], aggp[N:], degw, x, W1l, b1l, W1r)
    (agg2p,) = _seg_sum(h1, src, dst, with_deg=False)
    a_src, a_dst = _t2(agg2p[:N], agg2p[N:], degw, h1, W2l, b2l, W2r,
                       Wm1[:, :H], Wm1[:, H:2 * H])
    comb = _edge_combine(a_src, a_dst, src, dst)
    return _t3(comb, edge_attr, Wm1[:, 2 * H:], bm1, Wm2, bm2)


# speedup vs baseline: 6.1116x; 1.0018x over previous
import functools

import jax
import jax.numpy as jnp
from jax import lax
from jax.experimental import pallas as pl
from jax.experimental.pallas import tpu as pltpu
from jax.experimental.pallas import tpu_sc as plsc

N = 10000
E = 320000
H = 128
DE = 16

NC = 2
NS = 16
NW = NC * NS
EPW = E // NW
C = 80
NCHUNK = EPW // C
RPS_STEP = 624
RPS_SZ = 640

_F32 = jnp.float32


def _sc_mesh():
    return plsc.VectorSubcoreMesh(core_axis_name="c", subcore_axis_name="s")


_NZC = RPS_SZ // C


def _fill_rows(ref, rows, groups, val):
    def body(j, carry):
        for g in range(groups):
            ref[j, pl.ds(g * 16, 16)] = jnp.full((16,), val, _F32)
        return carry

    lax.fori_loop(0, rows, body, 0)


DROWS = 80
NBUF = 5

CS = 40
NCH_S = EPW // CS
NOUT_S = NCH_S // NBUF
CE = 80
NCH_E = EPW // CE
NOUT_E = NCH_E // NBUF

_OFFS_S = (0, 16, 24)
_OFFS_E = (0, 16, 32, 48, 64)


def _regcopy_chunk(dst2d, j, src1d, off, offs):
    for g in offs:
        dst2d[j, pl.ds(g, 16)] = src1d[pl.ds(off + g, 16)]


def _seg_sum(table, src, dst, with_deg):
    out_type = [jax.ShapeDtypeStruct((2 * N, H), _F32)]
    if with_deg:
        out_type.append(jax.ShapeDtypeStruct((NW, DROWS * H), _F32))

    scratch = [
        pltpu.VMEM((NBUF * CS,), jnp.int32),
        pltpu.VMEM((NBUF * CS,), jnp.int32),
        pltpu.VMEM((NBUF, CS), jnp.int32),
        pltpu.VMEM((NBUF, CS), jnp.int32),
        pltpu.VMEM((NBUF, CS, H), _F32),
        pltpu.VMEM_SHARED((N, H), _F32),
    ] + [pltpu.SemaphoreType.DMA] * (2 * NBUF)
    if with_deg:
        scratch.append(pltpu.VMEM((DROWS * H + 16,), _F32))

    @functools.partial(
        pl.kernel,
        out_type=out_type,
        mesh=_sc_mesh(),
        scratch_types=scratch,
    )
    def k(*refs):
        if with_deg:
            (x_hbm, src_hbm, dst_hbm, agg_hbm, deg_hbm,
             sblk_v, dblk_v, srcc_v, dstc_v, rows_v, acc_sh,
             *sems, dloc_v) = refs
        else:
            (x_hbm, src_hbm, dst_hbm, agg_hbm,
             sblk_v, dblk_v, srcc_v, dstc_v, rows_v, acc_sh,
             *sems) = refs
        gsem = sems[:NBUF]
        ssem = sems[NBUF:2 * NBUF]
        cid = lax.axis_index("c")
        sid = lax.axis_index("s")
        wid = sid * NC + cid
        r0 = sid * RPS_STEP
        _fill_rows(rows_v.at[0], CS, H // 16, 0.0)
        zcopies = []
        for j in range(RPS_SZ // CS):
            if j >= NBUF:
                zcopies[j - NBUF].wait()
            zcopies.append(pltpu.async_copy(
                rows_v.at[0], acc_sh.at[pl.ds(r0 + j * CS, CS)],
                sems[j % NBUF]))
        for cp in zcopies[-NBUF:]:
            cp.wait()
        if with_deg:
            def zbody(j, carry):
                dloc_v[pl.ds(j * 16, 16)] = jnp.zeros((16,), _F32)
                return carry
            lax.fori_loop(0, DROWS * H // 16 + 1, zbody, 0)
        plsc.subcore_barrier()

        base = wid * EPW
        onehot0 = jnp.where(lax.iota(jnp.int32, 16) == 0, 1.0, 0.0)

        pltpu.sync_copy(src_hbm.at[pl.ds(base, NBUF * CS)], sblk_v)
        pltpu.sync_copy(dst_hbm.at[pl.ds(base, NBUF * CS)], dblk_v)
        for j in range(NBUF):
            _regcopy_chunk(srcc_v, j, sblk_v, j * CS, _OFFS_S)
            _regcopy_chunk(dstc_v, j, dblk_v, j * CS, _OFFS_S)
            pltpu.async_copy(x_hbm.at[srcc_v.at[j]], rows_v.at[j], gsem[j])

        def outer(o, carry):
            for j in range(NBUF):
                pltpu.make_async_copy(
                    x_hbm.at[srcc_v.at[j]], rows_v.at[j], gsem[j]).wait()
                pltpu.async_copy(rows_v.at[j], acc_sh.at[dstc_v.at[j]],
                                 ssem[j], add=True)
            if with_deg:
                for jc in range(NBUF):
                    for g2, lanes in ((0, range(16)), (16, range(16)),
                                      (24, range(8, 16))):
                        dvec = dstc_v[jc, pl.ds(g2, 16)]
                        for l in lanes:
                            dd = dvec[l]
                            cur = dloc_v[pl.ds(dd, 16)]
                            dloc_v[pl.ds(dd, 16)] = cur + onehot0
            @pl.when(o < NOUT_S - 1)
            def _next():
                nb = base + (o + 1) * NBUF * CS
                pltpu.sync_copy(src_hbm.at[pl.ds(nb, NBUF * CS)], sblk_v)
                pltpu.sync_copy(dst_hbm.at[pl.ds(nb, NBUF * CS)], dblk_v)
                for j in range(NBUF):
                    pltpu.make_async_copy(
                        rows_v.at[j], acc_sh.at[dstc_v.at[j]], ssem[j]).wait()
                    _regcopy_chunk(srcc_v, j, sblk_v, j * CS, _OFFS_S)
                    _regcopy_chunk(dstc_v, j, dblk_v, j * CS, _OFFS_S)
                    pltpu.async_copy(x_hbm.at[srcc_v.at[j]], rows_v.at[j],
                                     gsem[j])
            return carry

        lax.fori_loop(0, NOUT_S, outer, 0)
        for j in range(NBUF):
            pltpu.make_async_copy(
                rows_v.at[j], acc_sh.at[dstc_v.at[j]], ssem[j]).wait()
        plsc.subcore_barrier()
        out0 = cid * N + r0
        wcopies = [None] * NBUF
        for j in range(RPS_SZ // CS):
            sl = j % NBUF
            if wcopies[sl] is not None:
                wcopies[sl].wait()
            pltpu.sync_copy(acc_sh.at[pl.ds(r0 + j * CS, CS)], rows_v.at[sl])
            wcopies[sl] = pltpu.async_copy(
                rows_v.at[sl], agg_hbm.at[pl.ds(out0 + j * CS, CS)], sems[sl])
        for cp in wcopies:
            if cp is not None:
                cp.wait()
        if with_deg:
            pltpu.sync_copy(dloc_v.at[pl.ds(0, DROWS * H)], deg_hbm.at[wid])

    if with_deg:
        return k(table, src, dst)
    return k(table, src, dst)


def _edge_combine(a_src, a_dst, src, dst, e0, ne):
    epw = ne // NW
    nch = epw // CE
    nout = nch // NBUF

    @functools.partial(
        pl.kernel,
        out_type=jax.ShapeDtypeStruct((ne, H), _F32),
        mesh=_sc_mesh(),
        scratch_types=[
            pltpu.VMEM((epw,), jnp.int32),
            pltpu.VMEM((epw,), jnp.int32),
            pltpu.VMEM((NBUF, CE), jnp.int32),
            pltpu.VMEM((NBUF, CE), jnp.int32),
            pltpu.VMEM((NBUF, CE, H), _F32),
        ] + [pltpu.SemaphoreType.DMA] * (3 * NBUF),
    )
    def k(asrc_hbm, adst_hbm, src_hbm, dst_hbm, out_hbm,
          sidx_v, didx_v, srcc_v, dstc_v, r1_v, *sems):
        g1sem = sems[:NBUF]
        g2sem = sems[NBUF:2 * NBUF]
        wsem = sems[2 * NBUF:3 * NBUF]
        cid = lax.axis_index("c")
        sid = lax.axis_index("s")
        wid = sid * NC + cid
        base = wid * epw

        pltpu.sync_copy(src_hbm.at[pl.ds(e0 + base, epw)], sidx_v)
        pltpu.sync_copy(dst_hbm.at[pl.ds(e0 + base, epw)], didx_v)
        for j in range(NBUF):
            _regcopy_chunk(srcc_v, j, sidx_v, j * CE, _OFFS_E)
            _regcopy_chunk(dstc_v, j, didx_v, j * CE, _OFFS_E)
            pltpu.async_copy(asrc_hbm.at[srcc_v.at[j]], r1_v.at[j], g1sem[j])

        def outer(o, carry):
            for j in range(NBUF):
                pltpu.make_async_copy(
                    asrc_hbm.at[srcc_v.at[j]], r1_v.at[j], g1sem[j]).wait()
                pltpu.async_copy(adst_hbm.at[dstc_v.at[j]], r1_v.at[j],
                                 g2sem[j], add=True)
            for j in range(NBUF):
                off = base + (o * NBUF + j) * CE
                pltpu.make_async_copy(
                    adst_hbm.at[dstc_v.at[j]], r1_v.at[j], g2sem[j]).wait()
                pltpu.async_copy(r1_v.at[j], out_hbm.at[pl.ds(off, CE)],
                                 wsem[j])
            for j in range(NBUF):
                @pl.when(o < nout - 1)
                def _launch1(j=j):
                    off = base + (o * NBUF + j) * CE
                    pltpu.make_async_copy(
                        r1_v.at[j], out_hbm.at[pl.ds(off, CE)], wsem[j]).wait()
                    nxt = ((o + 1) * NBUF + j) * CE
                    _regcopy_chunk(srcc_v, j, sidx_v, nxt, _OFFS_E)
                    _regcopy_chunk(dstc_v, j, didx_v, nxt, _OFFS_E)
                    pltpu.async_copy(asrc_hbm.at[srcc_v.at[j]], r1_v.at[j],
                                     g1sem[j])
            return carry

        lax.fori_loop(0, nout, outer, 0)
        for j in range(NBUF):
            off = base + ((nout - 1) * NBUF + j) * CE
            pltpu.make_async_copy(
                r1_v.at[j], out_hbm.at[pl.ds(off, CE)], wsem[j]).wait()

    return k(a_src, a_dst, src, dst)


_RB = 1024
_NBLK = (N + _RB - 1) // _RB
_DR = _RB // H


def _dotT(a, b):
    return lax.dot_general(a, b, (((1,), (1,)), ((), ())),
                           preferred_element_type=_F32)


def _dot(a, b):
    return jnp.dot(a, b, preferred_element_type=_F32)


def _sage_block(a0, a1, dw, xin, wl, bl, wr, lm, mm, ones_h):
    deg8 = jnp.sum(dw[...], axis=0)
    recip8 = 1.0 / jnp.maximum(deg8, 1.0)
    rbcast = _dot(_dot(lm[...], recip8) * mm[...], ones_h[...])
    mean = (a0[...] + a1[...]) * rbcast
    h = _dotT(mean, wl[...]) + bl[...][None, :] + _dotT(xin[...], wr[...])
    return jnp.maximum(h, 0.0)


def _t1_body(a0, a1, dw, x_ref, wl, bl, wr, lm, mm, ones_h, out_ref):
    out_ref[...] = _sage_block(a0, a1, dw, x_ref, wl, bl, wr, lm, mm, ones_h)


def _t2_body(a0, a1, dw, h1_ref, wl, bl, wr, lm, mm, ones_h,
             ws, wd, asrc_ref, adst_ref):
    h2 = _sage_block(a0, a1, dw, h1_ref, wl, bl, wr, lm, mm, ones_h)
    asrc_ref[...] = _dotT(h2, ws[...])
    adst_ref[...] = _dotT(h2, wd[...])


def _node_specs():
    return [
        pl.BlockSpec((_RB, H), lambda i: (i, 0)),
        pl.BlockSpec((_RB, H), lambda i: (i, 0)),
        pl.BlockSpec((NW, _DR, H), lambda i: (0, i, 0)),
        pl.BlockSpec((_RB, H), lambda i: (i, 0)),
        pl.BlockSpec((H, H), lambda i: (0, 0)),
        pl.BlockSpec((H,), lambda i: (0,)),
        pl.BlockSpec((H, H), lambda i: (0, 0)),
        pl.BlockSpec((_RB, _DR), lambda i: (0, 0)),
        pl.BlockSpec((_RB, H), lambda i: (0, 0)),
        pl.BlockSpec((H, H), lambda i: (0, 0)),
    ]


def _unpack_consts():
    j = jnp.arange(_RB)
    lm = (j[:, None] // H == jnp.arange(_DR)[None, :]).astype(_F32)
    mm = (j[:, None] % H == jnp.arange(H)[None, :]).astype(_F32)
    ones_h = jnp.ones((H, H), _F32)
    return lm, mm, ones_h


def _t1(agg0, agg1, degw, x, wl, bl, wr):
    lm, mm, ones_h = _unpack_consts()
    return pl.pallas_call(
        _t1_body,
        grid=(_NBLK,),
        in_specs=_node_specs(),
        out_specs=pl.BlockSpec((_RB, H), lambda i: (i, 0)),
        out_shape=jax.ShapeDtypeStruct((N, H), _F32),
    )(agg0, agg1, degw, x, wl, bl, wr, lm, mm, ones_h)


def _t2(agg0, agg1, degw, h1, wl, bl, wr, ws, wd):
    lm, mm, ones_h = _unpack_consts()
    return pl.pallas_call(
        _t2_body,
        grid=(_NBLK,),
        in_specs=_node_specs() + [
            pl.BlockSpec((H, H), lambda i: (0, 0)),
            pl.BlockSpec((H, H), lambda i: (0, 0)),
        ],
        out_specs=[
            pl.BlockSpec((_RB, H), lambda i: (i, 0)),
            pl.BlockSpec((_RB, H), lambda i: (i, 0)),
        ],
        out_shape=[
            jax.ShapeDtypeStruct((N, H), _F32),
            jax.ShapeDtypeStruct((N, H), _F32),
        ],
    )(agg0, agg1, degw, h1, wl, bl, wr, lm, mm, ones_h, ws, wd)


_EB = 4096


def _t3_body(comb_ref, attr_ref, wattr_ref, bm1_ref, wm2_ref, bm2_ref, out_ref):
    ap = _dotT(attr_ref[...], wattr_ref[...])
    pre = comb_ref[...] + ap + bm1_ref[...][None, :]
    r = jnp.maximum(pre, 0.0)
    res = _dotT(wm2_ref[...], r)
    res = res + bm2_ref[...][:, None]
    out_ref[...] = jnp.reshape(res, (res.shape[1],))


def _t3(comb, edge_attr, wattr, bm1, wm2, bm2):
    ne = comb.shape[0]
    return pl.pallas_call(
        _t3_body,
        grid=((ne + _EB - 1) // _EB,),
        in_specs=[
            pl.BlockSpec((_EB, H), lambda i: (i, 0)),
            pl.BlockSpec((_EB, DE), lambda i: (i, 0)),
            pl.BlockSpec((H, DE), lambda i: (0, 0)),
            pl.BlockSpec((H,), lambda i: (0,)),
            pl.BlockSpec((1, H), lambda i: (0, 0)),
            pl.BlockSpec((1,), lambda i: (0,)),
        ],
        out_specs=pl.BlockSpec((_EB,), lambda i: (i,)),
        out_shape=jax.ShapeDtypeStruct((ne,), _F32),
    )(comb, edge_attr, wattr, bm1, wm2, bm2)


def kernel(x, edge_index, edge_attr, W1l, b1l, W1r, W2l, b2l, W2r,
           Wm1, bm1, Wm2, bm2):
    src = edge_index[0].astype(jnp.int32)
    dst = edge_index[1].astype(jnp.int32)

    aggp, degp = _seg_sum(x, src, dst, with_deg=True)
    degw = degp.reshape(NW, DROWS, H)
    h1 = _t1(aggp[:N], aggp[N:], degw, x, W1l, b1l, W1r)
    (agg2p,) = _seg_sum(h1, src, dst, with_deg=False)
    a_src, a_dst = _t2(agg2p[:N], agg2p[N:], degw, h1, W2l, b2l, W2r,
                       Wm1[:, :H], Wm1[:, H:2 * H])
    ea = 192000
    wattr = Wm1[:, 2 * H:]
    comb_a = _edge_combine(a_src, a_dst, src, dst, 0, ea)
    comb_b = _edge_combine(a_src, a_dst, src, dst, ea, E - ea)
    out_a = _t3(comb_a, edge_attr[:ea], wattr, bm1, Wm2, bm2)
    out_b = _t3(comb_b, edge_attr[ea:], wattr, bm1, Wm2, bm2)
    return jnp.concatenate([out_a, out_b])


# speedup vs baseline: 6.1628x; 1.0084x over previous
import functools

import jax
import jax.numpy as jnp
from jax import lax
from jax.experimental import pallas as pl
from jax.experimental.pallas import tpu as pltpu
from jax.experimental.pallas import tpu_sc as plsc

N = 10000
E = 320000
H = 128
DE = 16

NC = 2
NS = 16
NW = NC * NS
EPW = E // NW
C = 80
NCHUNK = EPW // C
RPS_STEP = 624
RPS_SZ = 640

_F32 = jnp.float32


def _sc_mesh():
    return plsc.VectorSubcoreMesh(core_axis_name="c", subcore_axis_name="s")


_NZC = RPS_SZ // C


def _fill_rows(ref, rows, groups, val):
    def body(j, carry):
        for g in range(groups):
            ref[j, pl.ds(g * 16, 16)] = jnp.full((16,), val, _F32)
        return carry

    lax.fori_loop(0, rows, body, 0)


DROWS = 80
NBUF = 5

CS = 40
NCH_S = EPW // CS
NOUT_S = NCH_S // NBUF
CE = 80
NCH_E = EPW // CE
NOUT_E = NCH_E // NBUF

_OFFS_S = (0, 16, 24)
_OFFS_E = (0, 16, 32, 48, 64)


def _regcopy_chunk(dst2d, j, src1d, off, offs):
    for g in offs:
        dst2d[j, pl.ds(g, 16)] = src1d[pl.ds(off + g, 16)]


def _seg_sum(table, src, dst, with_deg):
    out_type = [jax.ShapeDtypeStruct((2 * N, H), _F32)]
    if with_deg:
        out_type.append(jax.ShapeDtypeStruct((NW, DROWS * H), _F32))

    scratch = [
        pltpu.VMEM((NBUF * CS,), jnp.int32),
        pltpu.VMEM((NBUF * CS,), jnp.int32),
        pltpu.VMEM((NBUF, CS), jnp.int32),
        pltpu.VMEM((NBUF, CS), jnp.int32),
        pltpu.VMEM((NBUF, CS, H), _F32),
        pltpu.VMEM_SHARED((N, H), _F32),
    ] + [pltpu.SemaphoreType.DMA] * (2 * NBUF)
    if with_deg:
        scratch.append(pltpu.VMEM((DROWS * H + 16,), _F32))

    @functools.partial(
        pl.kernel,
        out_type=out_type,
        mesh=_sc_mesh(),
        scratch_types=scratch,
    )
    def k(*refs):
        if with_deg:
            (x_hbm, src_hbm, dst_hbm, agg_hbm, deg_hbm,
             sblk_v, dblk_v, srcc_v, dstc_v, rows_v, acc_sh,
             *sems, dloc_v) = refs
        else:
            (x_hbm, src_hbm, dst_hbm, agg_hbm,
             sblk_v, dblk_v, srcc_v, dstc_v, rows_v, acc_sh,
             *sems) = refs
        gsem = sems[:NBUF]
        ssem = sems[NBUF:2 * NBUF]
        cid = lax.axis_index("c")
        sid = lax.axis_index("s")
        wid = sid * NC + cid
        r0 = sid * RPS_STEP
        _fill_rows(rows_v.at[0], CS, H // 16, 0.0)
        zcopies = []
        for j in range(RPS_SZ // CS):
            if j >= NBUF:
                zcopies[j - NBUF].wait()
            zcopies.append(pltpu.async_copy(
                rows_v.at[0], acc_sh.at[pl.ds(r0 + j * CS, CS)],
                sems[j % NBUF]))
        for cp in zcopies[-NBUF:]:
            cp.wait()
        if with_deg:
            def zbody(j, carry):
                dloc_v[pl.ds(j * 16, 16)] = jnp.zeros((16,), _F32)
                return carry
            lax.fori_loop(0, DROWS * H // 16 + 1, zbody, 0)
        plsc.subcore_barrier()

        base = wid * EPW
        onehot0 = jnp.where(lax.iota(jnp.int32, 16) == 0, 1.0, 0.0)

        pltpu.sync_copy(src_hbm.at[pl.ds(base, NBUF * CS)], sblk_v)
        pltpu.sync_copy(dst_hbm.at[pl.ds(base, NBUF * CS)], dblk_v)
        for j in range(NBUF):
            _regcopy_chunk(srcc_v, j, sblk_v, j * CS, _OFFS_S)
            _regcopy_chunk(dstc_v, j, dblk_v, j * CS, _OFFS_S)
            pltpu.async_copy(x_hbm.at[srcc_v.at[j]], rows_v.at[j], gsem[j])

        def outer(o, carry):
            for j in range(NBUF):
                pltpu.make_async_copy(
                    x_hbm.at[srcc_v.at[j]], rows_v.at[j], gsem[j]).wait()
                pltpu.async_copy(rows_v.at[j], acc_sh.at[dstc_v.at[j]],
                                 ssem[j], add=True)
            if with_deg:
                for jc in range(NBUF):
                    for g2, lanes in ((0, range(16)), (16, range(16)),
                                      (24, range(8, 16))):
                        dvec = dstc_v[jc, pl.ds(g2, 16)]
                        for l in lanes:
                            dd = dvec[l]
                            cur = dloc_v[pl.ds(dd, 16)]
                            dloc_v[pl.ds(dd, 16)] = cur + onehot0
            @pl.when(o < NOUT_S - 1)
            def _next():
                nb = base + (o + 1) * NBUF * CS
                pltpu.sync_copy(src_hbm.at[pl.ds(nb, NBUF * CS)], sblk_v)
                pltpu.sync_copy(dst_hbm.at[pl.ds(nb, NBUF * CS)], dblk_v)
                for j in range(NBUF):
                    pltpu.make_async_copy(
                        rows_v.at[j], acc_sh.at[dstc_v.at[j]], ssem[j]).wait()
                    _regcopy_chunk(srcc_v, j, sblk_v, j * CS, _OFFS_S)
                    _regcopy_chunk(dstc_v, j, dblk_v, j * CS, _OFFS_S)
                    pltpu.async_copy(x_hbm.at[srcc_v.at[j]], rows_v.at[j],
                                     gsem[j])
            return carry

        lax.fori_loop(0, NOUT_S, outer, 0)
        for j in range(NBUF):
            pltpu.make_async_copy(
                rows_v.at[j], acc_sh.at[dstc_v.at[j]], ssem[j]).wait()
        plsc.subcore_barrier()
        out0 = cid * N + r0
        wcopies = [None] * NBUF
        for j in range(RPS_SZ // CS):
            sl = j % NBUF
            if wcopies[sl] is not None:
                wcopies[sl].wait()
            pltpu.sync_copy(acc_sh.at[pl.ds(r0 + j * CS, CS)], rows_v.at[sl])
            wcopies[sl] = pltpu.async_copy(
                rows_v.at[sl], agg_hbm.at[pl.ds(out0 + j * CS, CS)], sems[sl])
        for cp in wcopies:
            if cp is not None:
                cp.wait()
        if with_deg:
            pltpu.sync_copy(dloc_v.at[pl.ds(0, DROWS * H)], deg_hbm.at[wid])

    if with_deg:
        return k(table, src, dst)
    return k(table, src, dst)


def _edge_combine(a_src, a_dst, src, dst, e0, ne):
    epw = ne // NW
    nch = epw // CE
    nout = nch // NBUF

    @functools.partial(
        pl.kernel,
        out_type=jax.ShapeDtypeStruct((ne, H), _F32),
        mesh=_sc_mesh(),
        scratch_types=[
            pltpu.VMEM((epw,), jnp.int32),
            pltpu.VMEM((epw,), jnp.int32),
            pltpu.VMEM((NBUF, CE), jnp.int32),
            pltpu.VMEM((NBUF, CE), jnp.int32),
            pltpu.VMEM((NBUF, CE, H), _F32),
        ] + [pltpu.SemaphoreType.DMA] * (3 * NBUF),
    )
    def k(asrc_hbm, adst_hbm, src_hbm, dst_hbm, out_hbm,
          sidx_v, didx_v, srcc_v, dstc_v, r1_v, *sems):
        g1sem = sems[:NBUF]
        g2sem = sems[NBUF:2 * NBUF]
        wsem = sems[2 * NBUF:3 * NBUF]
        cid = lax.axis_index("c")
        sid = lax.axis_index("s")
        wid = sid * NC + cid
        base = wid * epw

        pltpu.sync_copy(src_hbm.at[pl.ds(e0 + base, epw)], sidx_v)
        pltpu.sync_copy(dst_hbm.at[pl.ds(e0 + base, epw)], didx_v)
        for j in range(NBUF):
            _regcopy_chunk(srcc_v, j, sidx_v, j * CE, _OFFS_E)
            _regcopy_chunk(dstc_v, j, didx_v, j * CE, _OFFS_E)
            pltpu.async_copy(asrc_hbm.at[srcc_v.at[j]], r1_v.at[j], g1sem[j])

        def outer(o, carry):
            for j in range(NBUF):
                pltpu.make_async_copy(
                    asrc_hbm.at[srcc_v.at[j]], r1_v.at[j], g1sem[j]).wait()
                pltpu.async_copy(adst_hbm.at[dstc_v.at[j]], r1_v.at[j],
                                 g2sem[j], add=True)
            for j in range(NBUF):
                off = base + (o * NBUF + j) * CE
                pltpu.make_async_copy(
                    adst_hbm.at[dstc_v.at[j]], r1_v.at[j], g2sem[j]).wait()
                pltpu.async_copy(r1_v.at[j], out_hbm.at[pl.ds(off, CE)],
                                 wsem[j])
            for j in range(NBUF):
                @pl.when(o < nout - 1)
                def _launch1(j=j):
                    off = base + (o * NBUF + j) * CE
                    pltpu.make_async_copy(
                        r1_v.at[j], out_hbm.at[pl.ds(off, CE)], wsem[j]).wait()
                    nxt = ((o + 1) * NBUF + j) * CE
                    _regcopy_chunk(srcc_v, j, sidx_v, nxt, _OFFS_E)
                    _regcopy_chunk(dstc_v, j, didx_v, nxt, _OFFS_E)
                    pltpu.async_copy(asrc_hbm.at[srcc_v.at[j]], r1_v.at[j],
                                     g1sem[j])
            return carry

        lax.fori_loop(0, nout, outer, 0)
        for j in range(NBUF):
            off = base + ((nout - 1) * NBUF + j) * CE
            pltpu.make_async_copy(
                r1_v.at[j], out_hbm.at[pl.ds(off, CE)], wsem[j]).wait()

    return k(a_src, a_dst, src, dst)


_RB = 1024
_NBLK = (N + _RB - 1) // _RB
_DR = _RB // H


def _dotT(a, b):
    return lax.dot_general(a, b, (((1,), (1,)), ((), ())),
                           preferred_element_type=_F32)


def _dot(a, b):
    return jnp.dot(a, b, preferred_element_type=_F32)


def _sage_block(a0, a1, dw, xin, wl, bl, wr, lm, mm, ones_h):
    deg8 = jnp.sum(dw[...], axis=0)
    recip8 = 1.0 / jnp.maximum(deg8, 1.0)
    rbcast = _dot(_dot(lm[...], recip8) * mm[...], ones_h[...])
    mean = (a0[...] + a1[...]) * rbcast
    h = _dotT(mean, wl[...]) + bl[...][None, :] + _dotT(xin[...], wr[...])
    return jnp.maximum(h, 0.0)


def _t1_body(a0, a1, dw, x_ref, wl, bl, wr, lm, mm, ones_h, out_ref):
    out_ref[...] = _sage_block(a0, a1, dw, x_ref, wl, bl, wr, lm, mm, ones_h)


def _t2_body(a0, a1, dw, h1_ref, wl, bl, wr, lm, mm, ones_h,
             ws, wd, asrc_ref, adst_ref):
    h2 = _sage_block(a0, a1, dw, h1_ref, wl, bl, wr, lm, mm, ones_h)
    asrc_ref[...] = _dotT(h2, ws[...])
    adst_ref[...] = _dotT(h2, wd[...])


def _node_specs():
    return [
        pl.BlockSpec((_RB, H), lambda i: (i, 0)),
        pl.BlockSpec((_RB, H), lambda i: (i, 0)),
        pl.BlockSpec((NW, _DR, H), lambda i: (0, i, 0)),
        pl.BlockSpec((_RB, H), lambda i: (i, 0)),
        pl.BlockSpec((H, H), lambda i: (0, 0)),
        pl.BlockSpec((H,), lambda i: (0,)),
        pl.BlockSpec((H, H), lambda i: (0, 0)),
        pl.BlockSpec((_RB, _DR), lambda i: (0, 0)),
        pl.BlockSpec((_RB, H), lambda i: (0, 0)),
        pl.BlockSpec((H, H), lambda i: (0, 0)),
    ]


def _unpack_consts():
    j = jnp.arange(_RB)
    lm = (j[:, None] // H == jnp.arange(_DR)[None, :]).astype(_F32)
    mm = (j[:, None] % H == jnp.arange(H)[None, :]).astype(_F32)
    ones_h = jnp.ones((H, H), _F32)
    return lm, mm, ones_h


def _t1(agg0, agg1, degw, x, wl, bl, wr):
    lm, mm, ones_h = _unpack_consts()
    return pl.pallas_call(
        _t1_body,
        grid=(_NBLK,),
        in_specs=_node_specs(),
        out_specs=pl.BlockSpec((_RB, H), lambda i: (i, 0)),
        out_shape=jax.ShapeDtypeStruct((N, H), _F32),
    )(agg0, agg1, degw, x, wl, bl, wr, lm, mm, ones_h)


def _t2(agg0, agg1, degw, h1, wl, bl, wr, ws, wd):
    lm, mm, ones_h = _unpack_consts()
    return pl.pallas_call(
        _t2_body,
        grid=(_NBLK,),
        in_specs=_node_specs() + [
            pl.BlockSpec((H, H), lambda i: (0, 0)),
            pl.BlockSpec((H, H), lambda i: (0, 0)),
        ],
        out_specs=[
            pl.BlockSpec((_RB, H), lambda i: (i, 0)),
            pl.BlockSpec((_RB, H), lambda i: (i, 0)),
        ],
        out_shape=[
            jax.ShapeDtypeStruct((N, H), _F32),
            jax.ShapeDtypeStruct((N, H), _F32),
        ],
    )(agg0, agg1, degw, h1, wl, bl, wr, lm, mm, ones_h, ws, wd)


_EB = 4096


def _t3_body(comb_ref, attr_ref, wattr_ref, bm1_ref, wm2_ref, bm2_ref,
             out_ref):
    ap = _dotT(attr_ref[...], wattr_ref[...])
    pre = comb_ref[...] + ap + bm1_ref[...][None, :]
    r = jnp.maximum(pre, 0.0)
    res = _dotT(wm2_ref[...], r)
    res = res + bm2_ref[...][:, None]
    out_ref[...] = jnp.reshape(res, (res.shape[1],))


def _t3(comb, edge_attr, wattr, bm1, wm2, bm2):
    ne = comb.shape[0]
    return pl.pallas_call(
        _t3_body,
        grid=((ne + _EB - 1) // _EB,),
        in_specs=[
            pl.BlockSpec((_EB, H), lambda i: (i, 0)),
            pl.BlockSpec((_EB, DE), lambda i: (i, 0)),
            pl.BlockSpec((H, DE), lambda i: (0, 0)),
            pl.BlockSpec((H,), lambda i: (0,)),
            pl.BlockSpec((1, H), lambda i: (0, 0)),
            pl.BlockSpec((1,), lambda i: (0,)),
        ],
        out_specs=pl.BlockSpec((_EB,), lambda i: (i,)),
        out_shape=jax.ShapeDtypeStruct((ne,), _F32),
    )(comb, edge_attr, wattr, bm1, wm2, bm2)


def kernel(x, edge_index, edge_attr, W1l, b1l, W1r, W2l, b2l, W2r,
           Wm1, bm1, Wm2, bm2):
    src = edge_index[0].astype(jnp.int32)
    dst = edge_index[1].astype(jnp.int32)

    aggp, degp = _seg_sum(x, src, dst, with_deg=True)
    degw = degp.reshape(NW, DROWS, H)
    h1 = _t1(aggp[:N], aggp[N:], degw, x, W1l, b1l, W1r)
    (agg2p,) = _seg_sum(h1, src, dst, with_deg=False)
    a_src, a_dst = _t2(agg2p[:N], agg2p[N:], degw, h1, W2l, b2l, W2r,
                       Wm1[:, :H], Wm1[:, H:2 * H])
    comb = _edge_combine(a_src, a_dst, src, dst, 0, E)
    return _t3(comb, edge_attr, Wm1[:, 2 * H:], bm1, Wm2, bm2)


# speedup vs baseline: 6.5064x; 1.0557x over previous
import functools

import jax
import jax.numpy as jnp
from jax import lax
from jax.experimental import pallas as pl
from jax.experimental.pallas import tpu as pltpu
from jax.experimental.pallas import tpu_sc as plsc

N = 10000
E = 320000
H = 128
DE = 16

NC = 2
NS = 16
NW = NC * NS
EPW = E // NW
C = 80
NCHUNK = EPW // C
RPS_STEP = 624
RPS_SZ = 640

_F32 = jnp.float32


def _sc_mesh():
    return plsc.VectorSubcoreMesh(core_axis_name="c", subcore_axis_name="s")


_NZC = RPS_SZ // C


def _fill_rows(ref, rows, groups, val):
    def body(j, carry):
        for g in range(groups):
            ref[j, pl.ds(g * 16, 16)] = jnp.full((16,), val, _F32)
        return carry

    lax.fori_loop(0, rows, body, 0)


DROWS = 80
NBUF = 5

CS = 40
NCH_S = EPW // CS
NOUT_S = NCH_S // NBUF
CE = 80
NCH_E = EPW // CE
NOUT_E = NCH_E // NBUF

_OFFS_S = (0, 16, 24)
_OFFS_E = (0, 16, 32, 48, 64)


def _regcopy_chunk(dst2d, j, src1d, off, offs):
    for g in offs:
        dst2d[j, pl.ds(g, 16)] = src1d[pl.ds(off + g, 16)]


def _seg_sum(table, src, dst, with_deg):
    out_type = [jax.ShapeDtypeStruct((2 * N, H), _F32)]
    if with_deg:
        out_type.append(jax.ShapeDtypeStruct((NW, DROWS * H), _F32))

    scratch = [
        pltpu.VMEM((NBUF * CS,), jnp.int32),
        pltpu.VMEM((NBUF * CS,), jnp.int32),
        pltpu.VMEM((NBUF, CS), jnp.int32),
        pltpu.VMEM((NBUF, CS), jnp.int32),
        pltpu.VMEM((NBUF, CS, H), _F32),
        pltpu.VMEM_SHARED((N, H), _F32),
    ] + [pltpu.SemaphoreType.DMA] * (2 * NBUF)
    if with_deg:
        scratch.append(pltpu.VMEM((DROWS * H + 16,), _F32))

    @functools.partial(
        pl.kernel,
        out_type=out_type,
        mesh=_sc_mesh(),
        scratch_types=scratch,
    )
    def k(*refs):
        if with_deg:
            (x_hbm, src_hbm, dst_hbm, agg_hbm, deg_hbm,
             sblk_v, dblk_v, srcc_v, dstc_v, rows_v, acc_sh,
             *sems, dloc_v) = refs
        else:
            (x_hbm, src_hbm, dst_hbm, agg_hbm,
             sblk_v, dblk_v, srcc_v, dstc_v, rows_v, acc_sh,
             *sems) = refs
        gsem = sems[:NBUF]
        ssem = sems[NBUF:2 * NBUF]
        cid = lax.axis_index("c")
        sid = lax.axis_index("s")
        wid = sid * NC + cid
        r0 = sid * RPS_STEP
        _fill_rows(rows_v.at[0], CS, H // 16, 0.0)
        zcopies = []
        for j in range(RPS_SZ // CS):
            if j >= NBUF:
                zcopies[j - NBUF].wait()
            zcopies.append(pltpu.async_copy(
                rows_v.at[0], acc_sh.at[pl.ds(r0 + j * CS, CS)],
                sems[j % NBUF]))
        for cp in zcopies[-NBUF:]:
            cp.wait()
        if with_deg:
            def zbody(j, carry):
                dloc_v[pl.ds(j * 16, 16)] = jnp.zeros((16,), _F32)
                return carry
            lax.fori_loop(0, DROWS * H // 16 + 1, zbody, 0)
        plsc.subcore_barrier()

        base = wid * EPW
        onehot0 = jnp.where(lax.iota(jnp.int32, 16) == 0, 1.0, 0.0)

        pltpu.sync_copy(src_hbm.at[pl.ds(base, NBUF * CS)], sblk_v)
        pltpu.sync_copy(dst_hbm.at[pl.ds(base, NBUF * CS)], dblk_v)
        for j in range(NBUF):
            _regcopy_chunk(srcc_v, j, sblk_v, j * CS, _OFFS_S)
            _regcopy_chunk(dstc_v, j, dblk_v, j * CS, _OFFS_S)
            pltpu.async_copy(x_hbm.at[srcc_v.at[j]], rows_v.at[j], gsem[j])

        def outer(o, carry):
            if with_deg:
                for jc in range(NBUF):
                    for g2, lanes in ((0, range(16)), (16, range(16)),
                                      (24, range(8, 16))):
                        dvec = dstc_v[jc, pl.ds(g2, 16)]
                        for l in lanes:
                            dd = dvec[l]
                            cur = dloc_v[pl.ds(dd, 16)]
                            dloc_v[pl.ds(dd, 16)] = cur + onehot0
            for j in range(NBUF):
                pltpu.make_async_copy(
                    x_hbm.at[srcc_v.at[j]], rows_v.at[j], gsem[j]).wait()
                pltpu.async_copy(rows_v.at[j], acc_sh.at[dstc_v.at[j]],
                                 ssem[j], add=True)
            @pl.when(o < NOUT_S - 1)
            def _next():
                nb = base + (o + 1) * NBUF * CS
                pltpu.sync_copy(src_hbm.at[pl.ds(nb, NBUF * CS)], sblk_v)
                pltpu.sync_copy(dst_hbm.at[pl.ds(nb, NBUF * CS)], dblk_v)
                for j in range(NBUF):
                    pltpu.make_async_copy(
                        rows_v.at[j], acc_sh.at[dstc_v.at[j]], ssem[j]).wait()
                    _regcopy_chunk(srcc_v, j, sblk_v, j * CS, _OFFS_S)
                    _regcopy_chunk(dstc_v, j, dblk_v, j * CS, _OFFS_S)
                    pltpu.async_copy(x_hbm.at[srcc_v.at[j]], rows_v.at[j],
                                     gsem[j])
            return carry

        lax.fori_loop(0, NOUT_S, outer, 0)
        for j in range(NBUF):
            pltpu.make_async_copy(
                rows_v.at[j], acc_sh.at[dstc_v.at[j]], ssem[j]).wait()
        plsc.subcore_barrier()
        out0 = cid * N + r0
        wcopies = [None] * NBUF
        for j in range(RPS_SZ // CS):
            sl = j % NBUF
            if wcopies[sl] is not None:
                wcopies[sl].wait()
            pltpu.sync_copy(acc_sh.at[pl.ds(r0 + j * CS, CS)], rows_v.at[sl])
            wcopies[sl] = pltpu.async_copy(
                rows_v.at[sl], agg_hbm.at[pl.ds(out0 + j * CS, CS)], sems[sl])
        for cp in wcopies:
            if cp is not None:
                cp.wait()
        if with_deg:
            pltpu.sync_copy(dloc_v.at[pl.ds(0, DROWS * H)], deg_hbm.at[wid])

    if with_deg:
        return k(table, src, dst)
    return k(table, src, dst)


def _edge_combine(a_src, a_dst, src, dst, e0, ne):
    epw = ne // NW
    nch = epw // CE
    nout = nch // NBUF

    @functools.partial(
        pl.kernel,
        out_type=jax.ShapeDtypeStruct((ne, H), _F32),
        mesh=_sc_mesh(),
        scratch_types=[
            pltpu.VMEM((epw,), jnp.int32),
            pltpu.VMEM((epw,), jnp.int32),
            pltpu.VMEM((NBUF, CE), jnp.int32),
            pltpu.VMEM((NBUF, CE), jnp.int32),
            pltpu.VMEM((NBUF, CE, H), _F32),
        ] + [pltpu.SemaphoreType.DMA] * (3 * NBUF),
    )
    def k(asrc_hbm, adst_hbm, src_hbm, dst_hbm, out_hbm,
          sidx_v, didx_v, srcc_v, dstc_v, r1_v, *sems):
        g1sem = sems[:NBUF]
        g2sem = sems[NBUF:2 * NBUF]
        wsem = sems[2 * NBUF:3 * NBUF]
        cid = lax.axis_index("c")
        sid = lax.axis_index("s")
        wid = sid * NC + cid
        base = wid * epw

        pltpu.sync_copy(src_hbm.at[pl.ds(e0 + base, epw)], sidx_v)
        pltpu.sync_copy(dst_hbm.at[pl.ds(e0 + base, epw)], didx_v)
        for j in range(NBUF):
            _regcopy_chunk(srcc_v, j, sidx_v, j * CE, _OFFS_E)
            _regcopy_chunk(dstc_v, j, didx_v, j * CE, _OFFS_E)
            pltpu.async_copy(asrc_hbm.at[srcc_v.at[j]], r1_v.at[j], g1sem[j])

        def outer(o, carry):
            for j in range(NBUF):
                pltpu.make_async_copy(
                    asrc_hbm.at[srcc_v.at[j]], r1_v.at[j], g1sem[j]).wait()
                pltpu.async_copy(adst_hbm.at[dstc_v.at[j]], r1_v.at[j],
                                 g2sem[j], add=True)
            for j in range(NBUF):
                off = base + (o * NBUF + j) * CE
                pltpu.make_async_copy(
                    adst_hbm.at[dstc_v.at[j]], r1_v.at[j], g2sem[j]).wait()
                pltpu.async_copy(r1_v.at[j], out_hbm.at[pl.ds(off, CE)],
                                 wsem[j])
            for j in range(NBUF):
                @pl.when(o < nout - 1)
                def _launch1(j=j):
                    off = base + (o * NBUF + j) * CE
                    pltpu.make_async_copy(
                        r1_v.at[j], out_hbm.at[pl.ds(off, CE)], wsem[j]).wait()
                    nxt = ((o + 1) * NBUF + j) * CE
                    _regcopy_chunk(srcc_v, j, sidx_v, nxt, _OFFS_E)
                    _regcopy_chunk(dstc_v, j, didx_v, nxt, _OFFS_E)
                    pltpu.async_copy(asrc_hbm.at[srcc_v.at[j]], r1_v.at[j],
                                     g1sem[j])
            return carry

        lax.fori_loop(0, nout, outer, 0)
        for j in range(NBUF):
            off = base + ((nout - 1) * NBUF + j) * CE
            pltpu.make_async_copy(
                r1_v.at[j], out_hbm.at[pl.ds(off, CE)], wsem[j]).wait()

    return k(a_src, a_dst, src, dst)


_RB = 1024
_NBLK = (N + _RB - 1) // _RB
_DR = _RB // H


def _dotT(a, b):
    return lax.dot_general(a, b, (((1,), (1,)), ((), ())),
                           preferred_element_type=_F32)


def _dot(a, b):
    return jnp.dot(a, b, preferred_element_type=_F32)


def _sage_block(a0, a1, dw, xin, wl, bl, wr, lm, mm, ones_h):
    deg8 = jnp.sum(dw[...], axis=0)
    recip8 = 1.0 / jnp.maximum(deg8, 1.0)
    rbcast = _dot(_dot(lm[...], recip8) * mm[...], ones_h[...])
    mean = (a0[...] + a1[...]) * rbcast
    h = _dotT(mean, wl[...]) + bl[...][None, :] + _dotT(xin[...], wr[...])
    return jnp.maximum(h, 0.0)


def _t1_body(a0, a1, dw, x_ref, wl, bl, wr, lm, mm, ones_h, out_ref):
    out_ref[...] = _sage_block(a0, a1, dw, x_ref, wl, bl, wr, lm, mm, ones_h)


def _t2_body(a0, a1, dw, h1_ref, wl, bl, wr, lm, mm, ones_h,
             ws, wd, asrc_ref, adst_ref):
    h2 = _sage_block(a0, a1, dw, h1_ref, wl, bl, wr, lm, mm, ones_h)
    asrc_ref[...] = _dotT(h2, ws[...])
    adst_ref[...] = _dotT(h2, wd[...])


def _node_specs():
    return [
        pl.BlockSpec((_RB, H), lambda i: (i, 0)),
        pl.BlockSpec((_RB, H), lambda i: (i, 0)),
        pl.BlockSpec((NW, _DR, H), lambda i: (0, i, 0)),
        pl.BlockSpec((_RB, H), lambda i: (i, 0)),
        pl.BlockSpec((H, H), lambda i: (0, 0)),
        pl.BlockSpec((H,), lambda i: (0,)),
        pl.BlockSpec((H, H), lambda i: (0, 0)),
        pl.BlockSpec((_RB, _DR), lambda i: (0, 0)),
        pl.BlockSpec((_RB, H), lambda i: (0, 0)),
        pl.BlockSpec((H, H), lambda i: (0, 0)),
    ]


def _unpack_consts():
    j = jnp.arange(_RB)
    lm = (j[:, None] // H == jnp.arange(_DR)[None, :]).astype(_F32)
    mm = (j[:, None] % H == jnp.arange(H)[None, :]).astype(_F32)
    ones_h = jnp.ones((H, H), _F32)
    return lm, mm, ones_h


def _t1(agg0, agg1, degw, x, wl, bl, wr):
    lm, mm, ones_h = _unpack_consts()
    return pl.pallas_call(
        _t1_body,
        grid=(_NBLK,),
        in_specs=_node_specs(),
        out_specs=pl.BlockSpec((_RB, H), lambda i: (i, 0)),
        out_shape=jax.ShapeDtypeStruct((N, H), _F32),
    )(agg0, agg1, degw, x, wl, bl, wr, lm, mm, ones_h)


def _t2(agg0, agg1, degw, h1, wl, bl, wr, ws, wd):
    lm, mm, ones_h = _unpack_consts()
    return pl.pallas_call(
        _t2_body,
        grid=(_NBLK,),
        in_specs=_node_specs() + [
            pl.BlockSpec((H, H), lambda i: (0, 0)),
            pl.BlockSpec((H, H), lambda i: (0, 0)),
        ],
        out_specs=[
            pl.BlockSpec((_RB, H), lambda i: (i, 0)),
            pl.BlockSpec((_RB, H), lambda i: (i, 0)),
        ],
        out_shape=[
            jax.ShapeDtypeStruct((N, H), _F32),
            jax.ShapeDtypeStruct((N, H), _F32),
        ],
    )(agg0, agg1, degw, h1, wl, bl, wr, lm, mm, ones_h, ws, wd)


_EB = 4096


def _t3_body(comb_ref, attr_ref, wattr_ref, bm1_ref, wm2_ref, bm2_ref,
             out_ref):
    ap = _dotT(attr_ref[...], wattr_ref[...])
    pre = comb_ref[...] + ap + bm1_ref[...][None, :]
    r = jnp.maximum(pre, 0.0)
    res = _dotT(wm2_ref[...], r)
    res = res + bm2_ref[...][:, None]
    out_ref[...] = jnp.reshape(res, (res.shape[1],))


def _t3(comb, edge_attr, wattr, bm1, wm2, bm2):
    ne = comb.shape[0]
    return pl.pallas_call(
        _t3_body,
        grid=((ne + _EB - 1) // _EB,),
        in_specs=[
            pl.BlockSpec((_EB, H), lambda i: (i, 0)),
            pl.BlockSpec((_EB, DE), lambda i: (i, 0)),
            pl.BlockSpec((H, DE), lambda i: (0, 0)),
            pl.BlockSpec((H,), lambda i: (0,)),
            pl.BlockSpec((1, H), lambda i: (0, 0)),
            pl.BlockSpec((1,), lambda i: (0,)),
        ],
        out_specs=pl.BlockSpec((_EB,), lambda i: (i,)),
        out_shape=jax.ShapeDtypeStruct((ne,), _F32),
    )(comb, edge_attr, wattr, bm1, wm2, bm2)


def kernel(x, edge_index, edge_attr, W1l, b1l, W1r, W2l, b2l, W2r,
           Wm1, bm1, Wm2, bm2):
    src = edge_index[0].astype(jnp.int32)
    dst = edge_index[1].astype(jnp.int32)

    aggp, degp = _seg_sum(x, src, dst, with_deg=True)
    degw = degp.reshape(NW, DROWS, H)
    h1 = _t1(aggp[:N], aggp[N:], degw, x, W1l, b1l, W1r)
    (agg2p,) = _seg_sum(h1, src, dst, with_deg=False)
    a_src, a_dst = _t2(agg2p[:N], agg2p[N:], degw, h1, W2l, b2l, W2r,
                       Wm1[:, :H], Wm1[:, H:2 * H])
    comb = _edge_combine(a_src, a_dst, src, dst, 0, E)
    return _t3(comb, edge_attr, Wm1[:, 2 * H:], bm1, Wm2, bm2)
